# Initial kernel scaffold; baseline (speedup 1.0000x reference)
import jax
import jax.numpy as jnp
from jax.experimental import pallas as pl


def kernel(node_attr, edge_index, edge_attr, params):
    raise NotImplementedError("write your pallas kernel here")


# speedup vs baseline: 7.5181x; 7.5181x over previous
import functools

import jax
import jax.numpy as jnp
from jax import lax
from jax.experimental import pallas as pl
from jax.experimental.pallas import tpu as pltpu
from jax.experimental.pallas import tpu_sc as plsc

N = 10000
E = 160000
H = 128
NP = 10240
EP = 163840
C = 64
NCHUNK = EP // C
NWORK = 32
CPW = NCHUNK // NWORK
RB = 512
NB = NP // RB
NPT = NP // 16
NS = NP // 128
F32 = jnp.float32


def _lk(x):
    return jnp.where(x >= 0, x, 0.01 * x)


def _elu(x):
    return jnp.where(x > 0, x, jnp.exp(x) - 1.0)


def _gru_block(h, hid, wihT, whhT, bih, bhh):
    gi = jnp.dot(h, wihT, preferred_element_type=F32) + bih
    gh = jnp.dot(hid, whhT, preferred_element_type=F32) + bhh
    r = jax.nn.sigmoid(gi[:, :H] + gh[:, :H])
    z = jax.nn.sigmoid(gi[:, H:2 * H] + gh[:, H:2 * H])
    nn_ = jnp.tanh(gi[:, 2 * H:] + r * gh[:, 2 * H:])
    return (1.0 - z) * nn_ + z * hid


def _tc1_body(na_ref, l1w_ref, l1b_ref, w1aT_ref, g2T_ref, attr_ref,
              x0_ref, a_ref, m_ref, sr_ref):
    x0 = _lk(na_ref[...] * l1w_ref[...] + l1b_ref[...])
    x0_ref[...] = x0
    a_ref[...] = jnp.dot(x0, w1aT_ref[...], preferred_element_type=F32)
    m_ref[...] = jnp.dot(x0, g2T_ref[...], preferred_element_type=F32)
    sr_ref[...] = jnp.dot(x0, attr_ref[...], preferred_element_type=F32)


def _tc1(na, l1w, l1b, w1aT, g2T, attr_col):
    full = lambda s: pl.BlockSpec(s, lambda i: (0,) * len(s))
    return pl.pallas_call(
        _tc1_body,
        grid=(NB,),
        in_specs=[
            pl.BlockSpec((RB, 1), lambda i: (i, 0)),
            full((1, H)), full((1, H)), full((H, H)), full((H, H)),
            full((H, 1)),
        ],
        out_specs=[
            pl.BlockSpec((RB, H), lambda i: (i, 0)),
            pl.BlockSpec((RB, H), lambda i: (i, 0)),
            pl.BlockSpec((RB, H), lambda i: (i, 0)),
            pl.BlockSpec((RB, 1), lambda i: (i, 0)),
        ],
        out_shape=[
            jax.ShapeDtypeStruct((NP, H), F32),
            jax.ShapeDtypeStruct((NP, H), F32),
            jax.ShapeDtypeStruct((NP, H), F32),
            jax.ShapeDtypeStruct((NP, 1), F32),
        ],
        compiler_params=pltpu.CompilerParams(
            dimension_semantics=("arbitrary",)),
    )(na, l1w, l1b, w1aT, g2T, attr_col)


def _tc2_body(hacc_ref, s_ref, x0_ref, gb_ref, wihT_ref, whhT_ref, bih_ref,
              bhh_ref, awT_ref, asrc_ref, adst_ref,
              x1_ref, xs_ref, ssrc_ref, sdst_ref):
    hs = hacc_ref[0] + hacc_ref[1]
    ssum = s_ref[0] + s_ref[1]
    h = _elu(hs / (ssum + 1e-16) + gb_ref[...])
    x0 = x0_ref[...]
    x1 = jnp.maximum(
        _gru_block(h, x0, wihT_ref[...], whhT_ref[...], bih_ref[...],
                   bhh_ref[...]), 0.0)
    x1_ref[...] = x1
    xs = jnp.dot(x1, awT_ref[...], preferred_element_type=F32)
    xs_ref[...] = xs
    ssrc_ref[...] = jnp.dot(xs, asrc_ref[...], preferred_element_type=F32)
    sdst_ref[...] = jnp.dot(xs, adst_ref[...], preferred_element_type=F32)


def _tc2(hacc, s3, x0, gb, wihT, whhT, bih, bhh, awT, asrc_col, adst_col):
    full = lambda s: pl.BlockSpec(s, lambda i: (0,) * len(s))
    return pl.pallas_call(
        _tc2_body,
        grid=(NB,),
        in_specs=[
            pl.BlockSpec((2, RB, H), lambda i: (0, i, 0)),
            pl.BlockSpec((2, RB, 1), lambda i: (0, i, 0)),
            pl.BlockSpec((RB, H), lambda i: (i, 0)),
            full((1, H)), full((H, 3 * H)), full((H, 3 * H)),
            full((1, 3 * H)), full((1, 3 * H)), full((H, H)),
            full((H, 1)), full((H, 1)),
        ],
        out_specs=[
            pl.BlockSpec((RB, H), lambda i: (i, 0)),
            pl.BlockSpec((RB, H), lambda i: (i, 0)),
            pl.BlockSpec((RB, 1), lambda i: (i, 0)),
            pl.BlockSpec((RB, 1), lambda i: (i, 0)),
        ],
        out_shape=[
            jax.ShapeDtypeStruct((NP, H), F32),
            jax.ShapeDtypeStruct((NP, H), F32),
            jax.ShapeDtypeStruct((NP, 1), F32),
            jax.ShapeDtypeStruct((NP, 1), F32),
        ],
        compiler_params=pltpu.CompilerParams(
            dimension_semantics=("arbitrary",)),
    )(hacc, s3, x0, gb, wihT, whhT, bih, bhh, awT, asrc_col, adst_col)


def _tc3a_body(hacc_ref, s_ref, x1_ref, ab_ref, wihT_ref, whhT_ref, bih_ref,
               bhh_ref, mwT_ref, msrc_ref,
               xm_ref, ssrcm_ref, x2sum_ref, maxs_ref, acc_ref, mx_ref):
    i = pl.program_id(0)

    @pl.when(i == 0)
    def _():
        acc_ref[...] = jnp.zeros_like(acc_ref)
        mx_ref[0, 0] = -1e30

    hs = hacc_ref[0] + hacc_ref[1]
    ssum = s_ref[0] + s_ref[1]
    h = _elu(hs / (ssum + 1e-16) + ab_ref[...])
    x1 = x1_ref[...]
    x2 = jnp.maximum(
        _gru_block(h, x1, wihT_ref[...], whhT_ref[...], bih_ref[...],
                   bhh_ref[...]), 0.0)
    xm = jnp.dot(x2, mwT_ref[...], preferred_element_type=F32)
    xm_ref[...] = xm
    sm = jnp.dot(xm, msrc_ref[...], preferred_element_type=F32)
    rows = i * RB + lax.broadcasted_iota(jnp.int32, (RB, 1), 0)
    valid = rows < N
    sm = jnp.where(valid, sm, -1e30)
    ssrcm_ref[...] = sm
    acc_ref[...] += jnp.sum(jnp.where(valid, x2, 0.0), axis=0, keepdims=True)
    mx_ref[0, 0] = jnp.maximum(mx_ref[0, 0], jnp.max(sm))
    x2sum_ref[...] = acc_ref[...]
    maxs_ref[0, 0] = mx_ref[0, 0]


def _tc3a(hacc, s3, x1, ab, wihT, whhT, bih, bhh, mwT, msrc_col):
    full = lambda s: pl.BlockSpec(s, lambda i: (0,) * len(s))
    return pl.pallas_call(
        _tc3a_body,
        grid=(NB,),
        in_specs=[
            pl.BlockSpec((2, RB, H), lambda i: (0, i, 0)),
            pl.BlockSpec((2, RB, 1), lambda i: (0, i, 0)),
            pl.BlockSpec((RB, H), lambda i: (i, 0)),
            full((1, H)), full((H, 3 * H)), full((H, 3 * H)),
            full((1, 3 * H)), full((1, 3 * H)), full((H, H)),
            full((H, 1)),
        ],
        out_specs=[
            pl.BlockSpec((RB, H), lambda i: (i, 0)),
            pl.BlockSpec((RB, 1), lambda i: (i, 0)),
            pl.BlockSpec((1, H), lambda i: (0, 0)),
            pl.BlockSpec((1, 1), lambda i: (0, 0),
                         memory_space=pltpu.SMEM),
        ],
        out_shape=[
            jax.ShapeDtypeStruct((NP, H), F32),
            jax.ShapeDtypeStruct((NP, 1), F32),
            jax.ShapeDtypeStruct((1, H), F32),
            jax.ShapeDtypeStruct((1, 1), F32),
        ],
        scratch_shapes=[
            pltpu.VMEM((1, H), F32),
            pltpu.SMEM((1, 1), F32),
        ],
        compiler_params=pltpu.CompilerParams(
            dimension_semantics=("arbitrary",)),
    )(hacc, s3, x1, ab, wihT, whhT, bih, bhh, mwT, msrc_col)


def _tc3b_body(xm_ref, sm_ref, x2sum_ref, maxs_ref, mwT_ref, mdst_ref,
               mb_ref, wihT_ref, whhT_ref, bih_ref, bhh_ref, l2T_ref, l2b_ref,
               out_ref, sw_ref, wxm_ref):
    i = pl.program_id(0)

    @pl.when(i == 0)
    def _():
        sw_ref[0, 0] = 0.0
        wxm_ref[...] = jnp.zeros_like(wxm_ref)

    out0 = jnp.maximum(x2sum_ref[...], 0.0)
    cm = jnp.dot(out0, mwT_ref[...], preferred_element_type=F32)
    c = jnp.sum(cm * mdst_ref[...])
    mx = _lk(maxs_ref[0, 0] + c)
    am = _lk(sm_ref[...] + c)
    w = jnp.exp(am - mx)
    sw_ref[0, 0] += jnp.sum(w)
    wxm_ref[...] += jnp.sum(w * xm_ref[...], axis=0, keepdims=True)
    h3 = _elu(wxm_ref[...] / (sw_ref[0, 0] + 1e-16) + mb_ref[...])
    og = jnp.maximum(
        _gru_block(h3, out0, wihT_ref[...], whhT_ref[...], bih_ref[...],
                   bhh_ref[...]), 0.0)
    out_ref[...] = jnp.dot(og, l2T_ref[...], preferred_element_type=F32) \
        + l2b_ref[...]


def _tc3b(xm, ssrcm, x2sum, maxs, mwT, mdst_row, mb, wihT, whhT, bih, bhh,
          l2T, l2b):
    full = lambda s: pl.BlockSpec(s, lambda i: (0,) * len(s))
    return pl.pallas_call(
        _tc3b_body,
        grid=(NB,),
        in_specs=[
            pl.BlockSpec((RB, H), lambda i: (i, 0)),
            pl.BlockSpec((RB, 1), lambda i: (i, 0)),
            full((1, H)),
            pl.BlockSpec((1, 1), lambda i: (0, 0),
                         memory_space=pltpu.SMEM),
            full((H, H)), full((1, H)), full((1, H)),
            full((H, 3 * H)), full((H, 3 * H)),
            full((1, 3 * H)), full((1, 3 * H)),
            full((H, H)), full((1, H)),
        ],
        out_specs=pl.BlockSpec((1, H), lambda i: (0, 0)),
        out_shape=jax.ShapeDtypeStruct((1, H), F32),
        scratch_shapes=[
            pltpu.SMEM((1, 1), F32),
            pltpu.VMEM((1, H), F32),
        ],
        compiler_params=pltpu.CompilerParams(
            dimension_semantics=("arbitrary",)),
    )(xm, ssrcm, x2sum, maxs, mwT, mdst_row, mb, wihT, whhT, bih, bhh,
      l2T, l2b)


_MESH = dict(core_axis_name="c", subcore_axis_name="s",
             num_cores=2, num_subcores=16)


def _zero_rows(rows_ref, nrows):
    def zrow(e, _):
        for kk in range(H // 16):
            rows_ref[e, pl.ds(kk * 16, 16)] = jnp.zeros((16,), F32)
        return 0
    lax.fori_loop(0, nrows, zrow, 0)


def _prologue(rows_ref, s_part, rowidx, hacc, s_sp, sid):
    _zero_rows(rows_ref, C)
    _zero_rows(s_part, NS)
    for j in range(NPT // C):
        pltpu.sync_copy(rows_ref, hacc.at[pl.ds(sid * NPT + j * C, C)])

    @pl.when(sid < NS // 8)
    def _():
        pltpu.sync_copy(rows_ref.at[pl.ds(0, 8)], s_sp.at[pl.ds(sid * 8, 8)])
    lane = lax.iota(jnp.int32, 16)
    for j in range(NS // 16):
        rowidx[0, pl.ds(j * 16, 16)] = lane + j * 16


def _epilogue(sid, cid, s_part, rowidx, hacc, s_sp, hacc_out, s_out):
    pltpu.sync_copy(s_part, s_sp.at[rowidx.at[0]], add=True)
    plsc.subcore_barrier()
    for j in range(NPT // C):
        sl = pl.ds(sid * NPT + j * C, C)
        pltpu.sync_copy(hacc.at[sl], hacc_out.at[cid].at[sl])

    @pl.when(sid < NS // 8)
    def _():
        ssl = pl.ds(sid * 8, 8)
        pltpu.sync_copy(s_sp.at[ssl], s_out.at[cid].at[ssl])


def _sc1_body(a_hbm, m_hbm, sr_hbm, w1e_hbm, attl_hbm, src_hbm, dst_hbm,
              ea_hbm, hacc_out, s_out, hacc, s_sp, w1ev, attlv,
              s_part, rowidx, srcbuf, dstbuf, eabuf, srbuf, arows, mrows,
              tbuf, sem0, sem1, sem2):
    cid = lax.axis_index("c")
    sid = lax.axis_index("s")
    wid = cid * 16 + sid
    pltpu.sync_copy(w1e_hbm, w1ev)
    pltpu.sync_copy(attl_hbm, attlv)
    _prologue(arows, s_part, rowidx, hacc, s_sp, sid)
    plsc.subcore_barrier()
    w1 = [w1ev[pl.ds(kk * 16, 16)] for kk in range(8)]
    al = [attlv[pl.ds(kk * 16, 16)] for kk in range(8)]
    lane = lax.iota(jnp.int32, 16)

    def chunk(k, _):
        cidn = wid * CPW + k
        base = cidn * C
        pltpu.sync_copy(src_hbm.at[pl.ds(base, C)], srcbuf)
        pltpu.sync_copy(dst_hbm.at[pl.ds(cidn, 1)], dstbuf)
        pltpu.sync_copy(ea_hbm.at[pl.ds(base, C)], eabuf)
        d2 = pltpu.async_copy(sr_hbm.at[dstbuf.at[0]], srbuf, sem2)
        d0 = pltpu.async_copy(a_hbm.at[srcbuf], arows, sem0)
        d1 = pltpu.async_copy(m_hbm.at[srcbuf], mrows, sem1)
        d0.wait()

        def grp_t(g, _):
            ea16 = eabuf[pl.ds(g * 16, 16)]
            t16 = jnp.zeros((16,), F32)
            for e16 in range(16):
                e = g * 16 + e16
                ea_e = ea16[e16]
                acc = jnp.zeros((16,), F32)
                for kk in range(8):
                    v = arows[e, pl.ds(kk * 16, 16)] + ea_e * w1[kk]
                    v = jnp.where(v >= 0, v, 0.01 * v)
                    acc = acc + al[kk] * v
                t16 = jnp.where(lane == e16, jnp.sum(acc), t16)
            tbuf[pl.ds(g * 16, 16)] = t16
            return 0
        lax.fori_loop(0, C // 16, grp_t, 0)
        d2.wait()
        for g in range(C // 16):
            dst16 = dstbuf[0, pl.ds(g * 16, 16)]
            t16 = tbuf[pl.ds(g * 16, 16)] + srbuf[pl.ds(g * 16, 16)]
            t16 = jnp.where(t16 >= 0, t16, 0.01 * t16)
            ee16 = jnp.exp(t16)
            tbuf[pl.ds(g * 16, 16)] = ee16
            plsc.addupdate_scatter(s_part, [dst16 >> 7, dst16 & 127], ee16)
        d1.wait()

        def grp_s(g, _):
            ee16 = tbuf[pl.ds(g * 16, 16)]
            for e16 in range(16):
                e = g * 16 + e16
                ee_e = ee16[e16]
                for kk in range(8):
                    mrows[e, pl.ds(kk * 16, 16)] = \
                        mrows[e, pl.ds(kk * 16, 16)] * ee_e
            return 0
        lax.fori_loop(0, C // 16, grp_s, 0)
        pltpu.sync_copy(mrows, hacc.at[dstbuf.at[0]], add=True)
        return 0
    lax.fori_loop(0, CPW, chunk, 0)
    plsc.subcore_barrier()
    _epilogue(sid, cid, s_part, rowidx, hacc, s_sp, hacc_out, s_out)


@functools.cache
def _sc1_built():
    return pl.kernel(
        _sc1_body,
        out_type=[
            jax.ShapeDtypeStruct((2, NP, H), F32),
            jax.ShapeDtypeStruct((2, NS, 128), F32),
        ],
        mesh=plsc.VectorSubcoreMesh(**_MESH),
        scratch_types=[
            pltpu.VMEM_SHARED((NP, H), F32),
            pltpu.VMEM_SHARED((NS, 128), F32),
            pltpu.VMEM((H,), F32),
            pltpu.VMEM((H,), F32),
            pltpu.VMEM((NS, 128), F32),
            pltpu.VMEM((1, NS), jnp.int32),
            pltpu.VMEM((C,), jnp.int32),
            pltpu.VMEM((1, C), jnp.int32),
            pltpu.VMEM((C,), F32),
            pltpu.VMEM((C,), F32),
            pltpu.VMEM((C, H), F32),
            pltpu.VMEM((C, H), F32),
            pltpu.VMEM((C,), F32),
            pltpu.SemaphoreType.DMA,
            pltpu.SemaphoreType.DMA,
            pltpu.SemaphoreType.DMA,
        ],
        compiler_params=pltpu.CompilerParams(needs_layout_passes=False),
    )


def _sc1(*args):
    return _sc1_built()(*args)


def _sc2_body(xs_hbm, ssrc_hbm, sdst_hbm, src_hbm, dst_hbm, hacc_out, s_out,
              hacc, s_sp, s_part, rowidx, srcbuf, dstbuf, sabuf, sbbuf,
              xsrows, ebuf, sem0, sem1, sem2):
    cid = lax.axis_index("c")
    sid = lax.axis_index("s")
    wid = cid * 16 + sid
    _prologue(xsrows, s_part, rowidx, hacc, s_sp, sid)
    plsc.subcore_barrier()

    def chunk(k, _):
        cidn = wid * CPW + k
        base = cidn * C
        pltpu.sync_copy(src_hbm.at[pl.ds(base, C)], srcbuf)
        pltpu.sync_copy(dst_hbm.at[pl.ds(cidn, 1)], dstbuf)
        d1 = pltpu.async_copy(ssrc_hbm.at[srcbuf], sabuf, sem1)
        d2 = pltpu.async_copy(sdst_hbm.at[dstbuf.at[0]], sbbuf, sem2)
        d0 = pltpu.async_copy(xs_hbm.at[srcbuf], xsrows, sem0)
        d1.wait()
        d2.wait()
        for g in range(C // 16):
            dst16 = dstbuf[0, pl.ds(g * 16, 16)]
            a16 = sabuf[pl.ds(g * 16, 16)] + sbbuf[pl.ds(g * 16, 16)]
            a16 = jnp.where(a16 >= 0, a16, 0.01 * a16)
            ee16 = jnp.exp(a16)
            ebuf[pl.ds(g * 16, 16)] = ee16
            plsc.addupdate_scatter(s_part, [dst16 >> 7, dst16 & 127], ee16)
        d0.wait()

        def grp_s(g, _):
            ee16 = ebuf[pl.ds(g * 16, 16)]
            for e16 in range(16):
                e = g * 16 + e16
                ee_e = ee16[e16]
                for kk in range(8):
                    xsrows[e, pl.ds(kk * 16, 16)] = \
                        xsrows[e, pl.ds(kk * 16, 16)] * ee_e
            return 0
        lax.fori_loop(0, C // 16, grp_s, 0)
        pltpu.sync_copy(xsrows, hacc.at[dstbuf.at[0]], add=True)
        return 0
    lax.fori_loop(0, CPW, chunk, 0)
    plsc.subcore_barrier()
    _epilogue(sid, cid, s_part, rowidx, hacc, s_sp, hacc_out, s_out)


@functools.cache
def _sc2_built():
    return pl.kernel(
        _sc2_body,
        out_type=[
            jax.ShapeDtypeStruct((2, NP, H), F32),
            jax.ShapeDtypeStruct((2, NS, 128), F32),
        ],
        mesh=plsc.VectorSubcoreMesh(**_MESH),
        scratch_types=[
            pltpu.VMEM_SHARED((NP, H), F32),
            pltpu.VMEM_SHARED((NS, 128), F32),
            pltpu.VMEM((NS, 128), F32),
            pltpu.VMEM((1, NS), jnp.int32),
            pltpu.VMEM((C,), jnp.int32),
            pltpu.VMEM((1, C), jnp.int32),
            pltpu.VMEM((C,), F32),
            pltpu.VMEM((C,), F32),
            pltpu.VMEM((C, H), F32),
            pltpu.VMEM((C,), F32),
            pltpu.SemaphoreType.DMA,
            pltpu.SemaphoreType.DMA,
            pltpu.SemaphoreType.DMA,
        ],
        compiler_params=pltpu.CompilerParams(needs_layout_passes=False),
    )


def _sc2(*args):
    return _sc2_built()(*args)


def kernel(node_attr, edge_index, edge_attr, params):
    p = params
    na = jnp.pad(node_attr, ((0, NP - N), (0, 0)))
    src_p = jnp.pad(edge_index[0], (0, EP - E), constant_values=NP - 1)
    dst_p = jnp.pad(edge_index[1], (0, EP - E), constant_values=NP - 1)
    ea_p = jnp.pad(edge_attr[:, 0], (0, EP - E))
    dst2d = dst_p.reshape(NCHUNK, C)

    l1w = p["lin1_w"].T
    l1b = p["lin1_b"].reshape(1, H)
    w1aT = p["gate_lin1_w"][:, :H].T
    w1e = p["gate_lin1_w"][:, H]
    g2T = p["gate_lin2_w"].T
    attr_col = p["gate_att_r"].reshape(H, 1)

    x0, a_mat, m_mat, sr = _tc1(na, l1w, l1b, w1aT, g2T, attr_col)
    hacc, s1 = _sc1(a_mat, m_mat, sr.reshape(NP), w1e, p["gate_att_l"],
                    src_p, dst2d, ea_p)
    x1, xs, ssrc, sdst = _tc2(
        hacc, s1.reshape(2, NP, 1), x0, p["gate_bias"].reshape(1, H),
        p["gru1_wih"].T, p["gru1_whh"].T,
        p["gru1_bih"].reshape(1, 3 * H), p["gru1_bhh"].reshape(1, 3 * H),
        p["atom_w"].T, p["atom_att_src"].reshape(H, 1),
        p["atom_att_dst"].reshape(H, 1))
    hacc2, s2 = _sc2(xs, ssrc.reshape(NP), sdst.reshape(NP), src_p, dst2d)
    xm, ssrcm, x2sum, maxs = _tc3a(
        hacc2, s2.reshape(2, NP, 1), x1, p["atom_bias"].reshape(1, H),
        p["gru2_wih"].T, p["gru2_whh"].T,
        p["gru2_bih"].reshape(1, 3 * H), p["gru2_bhh"].reshape(1, 3 * H),
        p["mol_w"].T, p["mol_att_src"].reshape(H, 1))
    out = _tc3b(
        xm, ssrcm, x2sum, maxs, p["mol_w"].T,
        p["mol_att_dst"].reshape(1, H), p["mol_bias"].reshape(1, H),
        p["grum_wih"].T, p["grum_whh"].T,
        p["grum_bih"].reshape(1, 3 * H), p["grum_bhh"].reshape(1, 3 * H),
        p["lin2_w"].T, p["lin2_b"].reshape(1, H))
    return out


# speedup vs baseline: 9.8722x; 1.3131x over previous
import functools

import jax
import jax.numpy as jnp
from jax import lax
from jax.experimental import pallas as pl
from jax.experimental.pallas import tpu as pltpu
from jax.experimental.pallas import tpu_sc as plsc

N = 10000
E = 160000
H = 128
NP = 10240
EP = 163840
C = 32
BSZ = 8
NCHUNK = EP // C
NWORK = 32
CPW = NCHUNK // NWORK
RB = 512
NB = NP // RB
NPT = NP // 16
NS = NP // 128
F32 = jnp.float32


def _lk(x):
    return jnp.where(x >= 0, x, 0.01 * x)


def _elu(x):
    return jnp.where(x > 0, x, jnp.exp(x) - 1.0)


def _gru_block(h, hid, wihT, whhT, bih, bhh):
    gi = jnp.dot(h, wihT, preferred_element_type=F32) + bih
    gh = jnp.dot(hid, whhT, preferred_element_type=F32) + bhh
    r = jax.nn.sigmoid(gi[:, :H] + gh[:, :H])
    z = jax.nn.sigmoid(gi[:, H:2 * H] + gh[:, H:2 * H])
    nn_ = jnp.tanh(gi[:, 2 * H:] + r * gh[:, 2 * H:])
    return (1.0 - z) * nn_ + z * hid


def _tc1_body(na_ref, l1w_ref, l1b_ref, w1aT_ref, g2T_ref, attr_ref,
              x0_ref, a_ref, m_ref, sr_ref):
    x0 = _lk(na_ref[...] * l1w_ref[...] + l1b_ref[...])
    x0_ref[...] = x0
    a_ref[...] = jnp.dot(x0, w1aT_ref[...], preferred_element_type=F32)
    m_ref[...] = jnp.dot(x0, g2T_ref[...], preferred_element_type=F32)
    sr_ref[...] = jnp.dot(x0, attr_ref[...], preferred_element_type=F32)


def _tc1(na, l1w, l1b, w1aT, g2T, attr_col):
    full = lambda s: pl.BlockSpec(s, lambda i: (0,) * len(s))
    return pl.pallas_call(
        _tc1_body,
        grid=(NB,),
        in_specs=[
            pl.BlockSpec((RB, 1), lambda i: (i, 0)),
            full((1, H)), full((1, H)), full((H, H)), full((H, H)),
            full((H, 1)),
        ],
        out_specs=[
            pl.BlockSpec((RB, H), lambda i: (i, 0)),
            pl.BlockSpec((RB, H), lambda i: (i, 0)),
            pl.BlockSpec((RB, H), lambda i: (i, 0)),
            pl.BlockSpec((RB, 1), lambda i: (i, 0)),
        ],
        out_shape=[
            jax.ShapeDtypeStruct((NP, H), F32),
            jax.ShapeDtypeStruct((NP, H), F32),
            jax.ShapeDtypeStruct((NP, H), F32),
            jax.ShapeDtypeStruct((NP, 1), F32),
        ],
        compiler_params=pltpu.CompilerParams(
            dimension_semantics=("arbitrary",)),
    )(na, l1w, l1b, w1aT, g2T, attr_col)


def _tc2_body(hacc_ref, s_ref, x0_ref, gb_ref, wihT_ref, whhT_ref, bih_ref,
              bhh_ref, awT_ref, asrc_ref, adst_ref,
              x1_ref, xs_ref, ssrc_ref, sdst_ref):
    hs = hacc_ref[0] + hacc_ref[1]
    ssum = s_ref[0] + s_ref[1]
    h = _elu(hs / (ssum + 1e-16) + gb_ref[...])
    x0 = x0_ref[...]
    x1 = jnp.maximum(
        _gru_block(h, x0, wihT_ref[...], whhT_ref[...], bih_ref[...],
                   bhh_ref[...]), 0.0)
    x1_ref[...] = x1
    xs = jnp.dot(x1, awT_ref[...], preferred_element_type=F32)
    xs_ref[...] = xs
    ssrc_ref[...] = jnp.dot(xs, asrc_ref[...], preferred_element_type=F32)
    sdst_ref[...] = jnp.dot(xs, adst_ref[...], preferred_element_type=F32)


def _tc2(hacc, s3, x0, gb, wihT, whhT, bih, bhh, awT, asrc_col, adst_col):
    full = lambda s: pl.BlockSpec(s, lambda i: (0,) * len(s))
    return pl.pallas_call(
        _tc2_body,
        grid=(NB,),
        in_specs=[
            pl.BlockSpec((2, RB, H), lambda i: (0, i, 0)),
            pl.BlockSpec((2, RB, 1), lambda i: (0, i, 0)),
            pl.BlockSpec((RB, H), lambda i: (i, 0)),
            full((1, H)), full((H, 3 * H)), full((H, 3 * H)),
            full((1, 3 * H)), full((1, 3 * H)), full((H, H)),
            full((H, 1)), full((H, 1)),
        ],
        out_specs=[
            pl.BlockSpec((RB, H), lambda i: (i, 0)),
            pl.BlockSpec((RB, H), lambda i: (i, 0)),
            pl.BlockSpec((RB, 1), lambda i: (i, 0)),
            pl.BlockSpec((RB, 1), lambda i: (i, 0)),
        ],
        out_shape=[
            jax.ShapeDtypeStruct((NP, H), F32),
            jax.ShapeDtypeStruct((NP, H), F32),
            jax.ShapeDtypeStruct((NP, 1), F32),
            jax.ShapeDtypeStruct((NP, 1), F32),
        ],
        compiler_params=pltpu.CompilerParams(
            dimension_semantics=("arbitrary",)),
    )(hacc, s3, x0, gb, wihT, whhT, bih, bhh, awT, asrc_col, adst_col)


def _tc3a_body(hacc_ref, s_ref, x1_ref, ab_ref, wihT_ref, whhT_ref, bih_ref,
               bhh_ref, mwT_ref, msrc_ref,
               xm_ref, ssrcm_ref, x2sum_ref, maxs_ref, acc_ref, mx_ref):
    i = pl.program_id(0)

    @pl.when(i == 0)
    def _():
        acc_ref[...] = jnp.zeros_like(acc_ref)
        mx_ref[0, 0] = -1e30

    hs = hacc_ref[0] + hacc_ref[1]
    ssum = s_ref[0] + s_ref[1]
    h = _elu(hs / (ssum + 1e-16) + ab_ref[...])
    x1 = x1_ref[...]
    x2 = jnp.maximum(
        _gru_block(h, x1, wihT_ref[...], whhT_ref[...], bih_ref[...],
                   bhh_ref[...]), 0.0)
    xm = jnp.dot(x2, mwT_ref[...], preferred_element_type=F32)
    xm_ref[...] = xm
    sm = jnp.dot(xm, msrc_ref[...], preferred_element_type=F32)
    rows = i * RB + lax.broadcasted_iota(jnp.int32, (RB, 1), 0)
    valid = rows < N
    sm = jnp.where(valid, sm, -1e30)
    ssrcm_ref[...] = sm
    acc_ref[...] += jnp.sum(jnp.where(valid, x2, 0.0), axis=0, keepdims=True)
    mx_ref[0, 0] = jnp.maximum(mx_ref[0, 0], jnp.max(sm))
    x2sum_ref[...] = acc_ref[...]
    maxs_ref[0, 0] = mx_ref[0, 0]


def _tc3a(hacc, s3, x1, ab, wihT, whhT, bih, bhh, mwT, msrc_col):
    full = lambda s: pl.BlockSpec(s, lambda i: (0,) * len(s))
    return pl.pallas_call(
        _tc3a_body,
        grid=(NB,),
        in_specs=[
            pl.BlockSpec((2, RB, H), lambda i: (0, i, 0)),
            pl.BlockSpec((2, RB, 1), lambda i: (0, i, 0)),
            pl.BlockSpec((RB, H), lambda i: (i, 0)),
            full((1, H)), full((H, 3 * H)), full((H, 3 * H)),
            full((1, 3 * H)), full((1, 3 * H)), full((H, H)),
            full((H, 1)),
        ],
        out_specs=[
            pl.BlockSpec((RB, H), lambda i: (i, 0)),
            pl.BlockSpec((RB, 1), lambda i: (i, 0)),
            pl.BlockSpec((1, H), lambda i: (0, 0)),
            pl.BlockSpec((1, 1), lambda i: (0, 0),
                         memory_space=pltpu.SMEM),
        ],
        out_shape=[
            jax.ShapeDtypeStruct((NP, H), F32),
            jax.ShapeDtypeStruct((NP, 1), F32),
            jax.ShapeDtypeStruct((1, H), F32),
            jax.ShapeDtypeStruct((1, 1), F32),
        ],
        scratch_shapes=[
            pltpu.VMEM((1, H), F32),
            pltpu.SMEM((1, 1), F32),
        ],
        compiler_params=pltpu.CompilerParams(
            dimension_semantics=("arbitrary",)),
    )(hacc, s3, x1, ab, wihT, whhT, bih, bhh, mwT, msrc_col)


def _tc3b_body(xm_ref, sm_ref, x2sum_ref, maxs_ref, mwT_ref, mdst_ref,
               mb_ref, wihT_ref, whhT_ref, bih_ref, bhh_ref, l2T_ref, l2b_ref,
               out_ref, sw_ref, wxm_ref):
    i = pl.program_id(0)

    @pl.when(i == 0)
    def _():
        sw_ref[0, 0] = 0.0
        wxm_ref[...] = jnp.zeros_like(wxm_ref)

    out0 = jnp.maximum(x2sum_ref[...], 0.0)
    cm = jnp.dot(out0, mwT_ref[...], preferred_element_type=F32)
    c = jnp.sum(cm * mdst_ref[...])
    mx = _lk(maxs_ref[0, 0] + c)
    am = _lk(sm_ref[...] + c)
    w = jnp.exp(am - mx)
    sw_ref[0, 0] += jnp.sum(w)
    wxm_ref[...] += jnp.sum(w * xm_ref[...], axis=0, keepdims=True)
    h3 = _elu(wxm_ref[...] / (sw_ref[0, 0] + 1e-16) + mb_ref[...])
    og = jnp.maximum(
        _gru_block(h3, out0, wihT_ref[...], whhT_ref[...], bih_ref[...],
                   bhh_ref[...]), 0.0)
    out_ref[...] = jnp.dot(og, l2T_ref[...], preferred_element_type=F32) \
        + l2b_ref[...]


def _tc3b(xm, ssrcm, x2sum, maxs, mwT, mdst_row, mb, wihT, whhT, bih, bhh,
          l2T, l2b):
    full = lambda s: pl.BlockSpec(s, lambda i: (0,) * len(s))
    return pl.pallas_call(
        _tc3b_body,
        grid=(NB,),
        in_specs=[
            pl.BlockSpec((RB, H), lambda i: (i, 0)),
            pl.BlockSpec((RB, 1), lambda i: (i, 0)),
            full((1, H)),
            pl.BlockSpec((1, 1), lambda i: (0, 0),
                         memory_space=pltpu.SMEM),
            full((H, H)), full((1, H)), full((1, H)),
            full((H, 3 * H)), full((H, 3 * H)),
            full((1, 3 * H)), full((1, 3 * H)),
            full((H, H)), full((1, H)),
        ],
        out_specs=pl.BlockSpec((1, H), lambda i: (0, 0)),
        out_shape=jax.ShapeDtypeStruct((1, H), F32),
        scratch_shapes=[
            pltpu.SMEM((1, 1), F32),
            pltpu.VMEM((1, H), F32),
        ],
        compiler_params=pltpu.CompilerParams(
            dimension_semantics=("arbitrary",)),
    )(xm, ssrcm, x2sum, maxs, mwT, mdst_row, mb, wihT, whhT, bih, bhh,
      l2T, l2b)


_MESH = dict(core_axis_name="c", subcore_axis_name="s",
             num_cores=2, num_subcores=16)


def _zero_rows(rows_ref, nrows):
    def zrow(e, _):
        for kk in range(H // 16):
            rows_ref[e, pl.ds(kk * 16, 16)] = jnp.zeros((16,), F32)
        return 0
    lax.fori_loop(0, nrows, zrow, 0)


def _prologue(rows_ref, s_part, hacc, s_sp, sid):
    _zero_rows(rows_ref, C)
    _zero_rows(s_part, NS)

    def zh(j, _):
        pltpu.sync_copy(rows_ref, hacc.at[pl.ds(sid * NPT + j * C, C)])
        return 0
    lax.fori_loop(0, NPT // C, zh, 0)

    @pl.when(sid < NS // 8)
    def _():
        pltpu.sync_copy(rows_ref.at[pl.ds(0, 8)], s_sp.at[pl.ds(sid * 8, 8)])


def _epilogue(sid, cid, s_part, hacc, s_sp, hacc_out, s_out):
    lane = lax.iota(jnp.int32, 16)
    for j in range(NS // 16):
        pltpu.sync_copy(s_part.at[pl.ds(j * 16, 16)], s_sp.at[lane + j * 16],
                        add=True)
    plsc.subcore_barrier()

    def wh(j, _):
        sl = pl.ds(sid * NPT + j * C, C)
        pltpu.sync_copy(hacc.at[sl], hacc_out.at[cid].at[sl])
        return 0
    lax.fori_loop(0, NPT // C, wh, 0)

    @pl.when(sid < NS // 8)
    def _():
        ssl = pl.ds(sid * 8, 8)
        pltpu.sync_copy(s_sp.at[ssl], s_out.at[cid].at[ssl])


def _sc1_body(a_hbm, m_hbm, sr_hbm, w1e_hbm, attl_hbm, src_hbm, dst_hbm,
              ea_hbm, hacc_out, s_out, hacc, s_sp, w1ev, attlv,
              s_part, srcb, dstb, eab, srbuf, arows, mrows,
              tbuf, sem_a, sem_m, sem_sr, sem_sc):
    cid = lax.axis_index("c")
    sid = lax.axis_index("s")
    wid = cid * 16 + sid
    pltpu.sync_copy(w1e_hbm, w1ev)
    pltpu.sync_copy(attl_hbm, attlv)
    _prologue(arows.at[0], s_part, hacc, s_sp, sid)
    plsc.subcore_barrier()
    w1 = [w1ev[pl.ds(kk * 16, 16)] for kk in range(8)]
    al = [attlv[pl.ds(kk * 16, 16)] for kk in range(8)]
    lane = lax.iota(jnp.int32, 16)

    def load_batch(j, jb):
        row = wid * CPW + j * BSZ
        sl = pl.ds(jb * BSZ, BSZ)
        pltpu.sync_copy(src_hbm.at[pl.ds(row, BSZ)], srcb.at[sl])
        pltpu.sync_copy(dst_hbm.at[pl.ds(row, BSZ)], dstb.at[sl])
        pltpu.sync_copy(ea_hbm.at[pl.ds(row, BSZ)], eab.at[sl])

    def issue(kn):
        jbn = (kn // BSZ) & 1
        rown = jbn * BSZ + kn % BSZ
        bn = kn & 1
        bn3 = kn % 3
        pltpu.async_copy(sr_hbm.at[dstb.at[rown]], srbuf.at[bn],
                         sem_sr.at[bn])
        pltpu.async_copy(a_hbm.at[srcb.at[rown]], arows.at[bn], sem_a.at[bn])

        @pl.when(kn >= 3)
        def _():
            pltpu.make_async_copy(mrows.at[bn3], hacc.at[dstb.at[0]],
                                  sem_sc.at[bn3]).wait()
        pltpu.async_copy(m_hbm.at[srcb.at[rown]], mrows.at[bn3],
                         sem_m.at[bn3])

    load_batch(0, 0)
    issue(0)

    def chunk(k, _):
        b = k & 1
        b3 = k % 3
        row = ((k // BSZ) & 1) * BSZ + k % BSZ
        kn = k + 1

        @pl.when(kn < CPW)
        def _():
            @pl.when(kn % BSZ == 0)
            def _():
                load_batch(kn // BSZ, (kn // BSZ) & 1)
            issue(kn)

        pltpu.make_async_copy(a_hbm.at[srcb.at[row]], arows.at[b],
                              sem_a.at[b]).wait()
        pltpu.make_async_copy(sr_hbm.at[dstb.at[row]], srbuf.at[b],
                              sem_sr.at[b]).wait()

        def grp_t(g, _):
            ea16 = eab[row, pl.ds(g * 16, 16)]
            t16 = jnp.zeros((16,), F32)
            for e16 in range(16):
                e = g * 16 + e16
                ea_e = ea16[e16]
                acc = jnp.zeros((16,), F32)
                for kk in range(8):
                    v = arows[b, e, pl.ds(kk * 16, 16)] + ea_e * w1[kk]
                    v = jnp.maximum(v, 0.01 * v)
                    acc = acc + al[kk] * v
                t16 = jnp.where(lane == e16, jnp.sum(acc), t16)
            dst16 = dstb[row, pl.ds(g * 16, 16)]
            t16 = t16 + srbuf[b, pl.ds(g * 16, 16)]
            t16 = jnp.maximum(t16, 0.01 * t16)
            ee16 = jnp.exp(t16)
            tbuf[pl.ds(g * 16, 16)] = ee16
            plsc.addupdate_scatter(s_part, [dst16 >> 7, dst16 & 127], ee16)
            return 0
        lax.fori_loop(0, C // 16, grp_t, 0)
        pltpu.make_async_copy(m_hbm.at[srcb.at[row]], mrows.at[b3],
                              sem_m.at[b3]).wait()

        def grp_s(g, _):
            ee16 = tbuf[pl.ds(g * 16, 16)]
            for e16 in range(16):
                e = g * 16 + e16
                ee_e = ee16[e16]
                for kk in range(8):
                    mrows[b3, e, pl.ds(kk * 16, 16)] = \
                        mrows[b3, e, pl.ds(kk * 16, 16)] * ee_e
            return 0
        lax.fori_loop(0, C // 16, grp_s, 0)
        pltpu.async_copy(mrows.at[b3], hacc.at[dstb.at[row]], sem_sc.at[b3],
                         add=True)
        return 0
    lax.fori_loop(0, CPW, chunk, 0)
    for j in range(3):
        pltpu.make_async_copy(mrows.at[(CPW - 3 + j) % 3],
                              hacc.at[dstb.at[0]],
                              sem_sc.at[(CPW - 3 + j) % 3]).wait()
    plsc.subcore_barrier()
    _epilogue(sid, cid, s_part, hacc, s_sp, hacc_out, s_out)


@functools.cache
def _sc1_built():
    return pl.kernel(
        _sc1_body,
        out_type=[
            jax.ShapeDtypeStruct((2, NP, H), F32),
            jax.ShapeDtypeStruct((2, NS, 128), F32),
        ],
        mesh=plsc.VectorSubcoreMesh(**_MESH),
        scratch_types=[
            pltpu.VMEM_SHARED((NP, H), F32),
            pltpu.VMEM_SHARED((NS, 128), F32),
            pltpu.VMEM((H,), F32),
            pltpu.VMEM((H,), F32),
            pltpu.VMEM((NS, 128), F32),
            pltpu.VMEM((2 * BSZ, C), jnp.int32),
            pltpu.VMEM((2 * BSZ, C), jnp.int32),
            pltpu.VMEM((2 * BSZ, C), F32),
            pltpu.VMEM((2, C), F32),
            pltpu.VMEM((2, C, H), F32),
            pltpu.VMEM((3, C, H), F32),
            pltpu.VMEM((C,), F32),
            pltpu.SemaphoreType.DMA((2,)),
            pltpu.SemaphoreType.DMA((3,)),
            pltpu.SemaphoreType.DMA((2,)),
            pltpu.SemaphoreType.DMA((3,)),
        ],
        compiler_params=pltpu.CompilerParams(needs_layout_passes=False),
    )


def _sc1(*args):
    return _sc1_built()(*args)


def _sc2_body(xs_hbm, ssrc_hbm, sdst_hbm, src_hbm, dst_hbm, hacc_out, s_out,
              hacc, s_sp, s_part, srcb, dstb, sabuf, sbbuf,
              xsrows, ebuf, sem_x, sem_a, sem_b, sem_sc):
    cid = lax.axis_index("c")
    sid = lax.axis_index("s")
    wid = cid * 16 + sid
    _prologue(xsrows.at[0], s_part, hacc, s_sp, sid)
    plsc.subcore_barrier()

    def load_batch(j, jb):
        row = wid * CPW + j * BSZ
        sl = pl.ds(jb * BSZ, BSZ)
        pltpu.sync_copy(src_hbm.at[pl.ds(row, BSZ)], srcb.at[sl])
        pltpu.sync_copy(dst_hbm.at[pl.ds(row, BSZ)], dstb.at[sl])

    def issue(kn):
        rown = ((kn // BSZ) & 1) * BSZ + kn % BSZ
        bn = kn & 1
        bn3 = kn % 3
        pltpu.async_copy(ssrc_hbm.at[srcb.at[rown]], sabuf.at[bn],
                         sem_a.at[bn])
        pltpu.async_copy(sdst_hbm.at[dstb.at[rown]], sbbuf.at[bn],
                         sem_b.at[bn])

        @pl.when(kn >= 3)
        def _():
            pltpu.make_async_copy(xsrows.at[bn3], hacc.at[dstb.at[0]],
                                  sem_sc.at[bn3]).wait()
        pltpu.async_copy(xs_hbm.at[srcb.at[rown]], xsrows.at[bn3],
                         sem_x.at[bn3])

    load_batch(0, 0)
    issue(0)

    def chunk(k, _):
        b = k & 1
        b3 = k % 3
        row = ((k // BSZ) & 1) * BSZ + k % BSZ
        kn = k + 1

        @pl.when(kn < CPW)
        def _():
            @pl.when(kn % BSZ == 0)
            def _():
                load_batch(kn // BSZ, (kn // BSZ) & 1)
            issue(kn)

        pltpu.make_async_copy(ssrc_hbm.at[srcb.at[row]], sabuf.at[b],
                              sem_a.at[b]).wait()
        pltpu.make_async_copy(sdst_hbm.at[dstb.at[row]], sbbuf.at[b],
                              sem_b.at[b]).wait()
        for g in range(C // 16):
            dst16 = dstb[row, pl.ds(g * 16, 16)]
            a16 = sabuf[b, pl.ds(g * 16, 16)] + sbbuf[b, pl.ds(g * 16, 16)]
            a16 = jnp.maximum(a16, 0.01 * a16)
            ee16 = jnp.exp(a16)
            ebuf[pl.ds(g * 16, 16)] = ee16
            plsc.addupdate_scatter(s_part, [dst16 >> 7, dst16 & 127], ee16)
        pltpu.make_async_copy(xs_hbm.at[srcb.at[row]], xsrows.at[b3],
                              sem_x.at[b3]).wait()

        def grp_s(g, _):
            ee16 = ebuf[pl.ds(g * 16, 16)]
            for e16 in range(16):
                e = g * 16 + e16
                ee_e = ee16[e16]
                for kk in range(8):
                    xsrows[b3, e, pl.ds(kk * 16, 16)] = \
                        xsrows[b3, e, pl.ds(kk * 16, 16)] * ee_e
            return 0
        lax.fori_loop(0, C // 16, grp_s, 0)
        pltpu.async_copy(xsrows.at[b3], hacc.at[dstb.at[row]], sem_sc.at[b3],
                         add=True)
        return 0
    lax.fori_loop(0, CPW, chunk, 0)
    for j in range(3):
        pltpu.make_async_copy(xsrows.at[(CPW - 3 + j) % 3],
                              hacc.at[dstb.at[0]],
                              sem_sc.at[(CPW - 3 + j) % 3]).wait()
    plsc.subcore_barrier()
    _epilogue(sid, cid, s_part, hacc, s_sp, hacc_out, s_out)


@functools.cache
def _sc2_built():
    return pl.kernel(
        _sc2_body,
        out_type=[
            jax.ShapeDtypeStruct((2, NP, H), F32),
            jax.ShapeDtypeStruct((2, NS, 128), F32),
        ],
        mesh=plsc.VectorSubcoreMesh(**_MESH),
        scratch_types=[
            pltpu.VMEM_SHARED((NP, H), F32),
            pltpu.VMEM_SHARED((NS, 128), F32),
            pltpu.VMEM((NS, 128), F32),
            pltpu.VMEM((2 * BSZ, C), jnp.int32),
            pltpu.VMEM((2 * BSZ, C), jnp.int32),
            pltpu.VMEM((2, C), F32),
            pltpu.VMEM((2, C), F32),
            pltpu.VMEM((3, C, H), F32),
            pltpu.VMEM((C,), F32),
            pltpu.SemaphoreType.DMA((3,)),
            pltpu.SemaphoreType.DMA((2,)),
            pltpu.SemaphoreType.DMA((2,)),
            pltpu.SemaphoreType.DMA((3,)),
        ],
        compiler_params=pltpu.CompilerParams(needs_layout_passes=False),
    )


def _sc2(*args):
    return _sc2_built()(*args)


def kernel(node_attr, edge_index, edge_attr, params):
    p = params
    na = jnp.pad(node_attr, ((0, NP - N), (0, 0)))
    src_p = jnp.pad(edge_index[0], (0, EP - E), constant_values=NP - 1)
    dst_p = jnp.pad(edge_index[1], (0, EP - E), constant_values=NP - 1)
    ea_p = jnp.pad(edge_attr[:, 0], (0, EP - E))
    src2d = src_p.reshape(NCHUNK, C)
    dst2d = dst_p.reshape(NCHUNK, C)
    ea2d = ea_p.reshape(NCHUNK, C)

    l1w = p["lin1_w"].T
    l1b = p["lin1_b"].reshape(1, H)
    w1aT = p["gate_lin1_w"][:, :H].T
    w1e = p["gate_lin1_w"][:, H]
    g2T = p["gate_lin2_w"].T
    attr_col = p["gate_att_r"].reshape(H, 1)

    x0, a_mat, m_mat, sr = _tc1(na, l1w, l1b, w1aT, g2T, attr_col)
    hacc, s1 = _sc1(a_mat, m_mat, sr.reshape(NP), w1e, p["gate_att_l"],
                    src2d, dst2d, ea2d)
    x1, xs, ssrc, sdst = _tc2(
        hacc, s1.reshape(2, NP, 1), x0, p["gate_bias"].reshape(1, H),
        p["gru1_wih"].T, p["gru1_whh"].T,
        p["gru1_bih"].reshape(1, 3 * H), p["gru1_bhh"].reshape(1, 3 * H),
        p["atom_w"].T, p["atom_att_src"].reshape(H, 1),
        p["atom_att_dst"].reshape(H, 1))
    hacc2, s2 = _sc2(xs, ssrc.reshape(NP), sdst.reshape(NP), src2d, dst2d)
    xm, ssrcm, x2sum, maxs = _tc3a(
        hacc2, s2.reshape(2, NP, 1), x1, p["atom_bias"].reshape(1, H),
        p["gru2_wih"].T, p["gru2_whh"].T,
        p["gru2_bih"].reshape(1, 3 * H), p["gru2_bhh"].reshape(1, 3 * H),
        p["mol_w"].T, p["mol_att_src"].reshape(H, 1))
    out = _tc3b(
        xm, ssrcm, x2sum, maxs, p["mol_w"].T,
        p["mol_att_dst"].reshape(1, H), p["mol_bias"].reshape(1, H),
        p["grum_wih"].T, p["grum_whh"].T,
        p["grum_bih"].reshape(1, 3 * H), p["grum_bhh"].reshape(1, 3 * H),
        p["lin2_w"].T, p["lin2_b"].reshape(1, H))
    return out


# speedup vs baseline: 10.1686x; 1.0300x over previous
import functools

import jax
import jax.numpy as jnp
from jax import lax
from jax.experimental import pallas as pl
from jax.experimental.pallas import tpu as pltpu
from jax.experimental.pallas import tpu_sc as plsc

N = 10000
E = 160000
H = 128
NP = 10240
EP = 163840
C1 = 32
C2 = 64
BSZ = 8
NWORK = 32
CPW1 = EP // C1 // NWORK
CPW2 = EP // C2 // NWORK
RB = 512
NB = NP // RB
NPT = NP // 16
NS = NP // 128
F32 = jnp.float32


def _lk(x):
    return jnp.where(x >= 0, x, 0.01 * x)


def _elu(x):
    return jnp.where(x > 0, x, jnp.exp(x) - 1.0)


def _gru_block(h, hid, wihT, whhT, bih, bhh):
    gi = jnp.dot(h, wihT, preferred_element_type=F32) + bih
    gh = jnp.dot(hid, whhT, preferred_element_type=F32) + bhh
    r = jax.nn.sigmoid(gi[:, :H] + gh[:, :H])
    z = jax.nn.sigmoid(gi[:, H:2 * H] + gh[:, H:2 * H])
    nn_ = jnp.tanh(gi[:, 2 * H:] + r * gh[:, 2 * H:])
    return (1.0 - z) * nn_ + z * hid


def _tc1_body(na_ref, l1w_ref, l1b_ref, w1aT_ref, g2T_ref, attr_ref,
              x0_ref, am_ref, sr_ref):
    x0 = _lk(na_ref[...] * l1w_ref[...] + l1b_ref[...])
    x0_ref[...] = x0
    am_ref[:, :H] = jnp.dot(x0, w1aT_ref[...], preferred_element_type=F32)
    am_ref[:, H:] = jnp.dot(x0, g2T_ref[...], preferred_element_type=F32)
    sr_ref[...] = jnp.dot(x0, attr_ref[...], preferred_element_type=F32)


def _tc1(na, l1w, l1b, w1aT, g2T, attr_col):
    full = lambda s: pl.BlockSpec(s, lambda i: (0,) * len(s))
    return pl.pallas_call(
        _tc1_body,
        grid=(NB,),
        in_specs=[
            pl.BlockSpec((RB, 1), lambda i: (i, 0)),
            full((1, H)), full((1, H)), full((H, H)), full((H, H)),
            full((H, 1)),
        ],
        out_specs=[
            pl.BlockSpec((RB, H), lambda i: (i, 0)),
            pl.BlockSpec((RB, 2 * H), lambda i: (i, 0)),
            pl.BlockSpec((RB, 1), lambda i: (i, 0)),
        ],
        out_shape=[
            jax.ShapeDtypeStruct((NP, H), F32),
            jax.ShapeDtypeStruct((NP, 2 * H), F32),
            jax.ShapeDtypeStruct((NP, 1), F32),
        ],
        compiler_params=pltpu.CompilerParams(
            dimension_semantics=("arbitrary",)),
    )(na, l1w, l1b, w1aT, g2T, attr_col)


def _tc2_body(hacc_ref, s_ref, x0_ref, gb_ref, wihT_ref, whhT_ref, bih_ref,
              bhh_ref, awT_ref, asrc_ref, adst_ref,
              x1_ref, xs_ref, ssrc_ref, sdst_ref):
    hs = hacc_ref[0] + hacc_ref[1]
    ssum = s_ref[0] + s_ref[1]
    h = _elu(hs / (ssum + 1e-16) + gb_ref[...])
    x0 = x0_ref[...]
    x1 = jnp.maximum(
        _gru_block(h, x0, wihT_ref[...], whhT_ref[...], bih_ref[...],
                   bhh_ref[...]), 0.0)
    x1_ref[...] = x1
    xs = jnp.dot(x1, awT_ref[...], preferred_element_type=F32)
    xs_ref[...] = xs
    ssrc_ref[...] = jnp.dot(xs, asrc_ref[...], preferred_element_type=F32)
    sdst_ref[...] = jnp.dot(xs, adst_ref[...], preferred_element_type=F32)


def _tc2(hacc, s3, x0, gb, wihT, whhT, bih, bhh, awT, asrc_col, adst_col):
    full = lambda s: pl.BlockSpec(s, lambda i: (0,) * len(s))
    return pl.pallas_call(
        _tc2_body,
        grid=(NB,),
        in_specs=[
            pl.BlockSpec((2, RB, H), lambda i: (0, i, 0)),
            pl.BlockSpec((2, RB, 1), lambda i: (0, i, 0)),
            pl.BlockSpec((RB, H), lambda i: (i, 0)),
            full((1, H)), full((H, 3 * H)), full((H, 3 * H)),
            full((1, 3 * H)), full((1, 3 * H)), full((H, H)),
            full((H, 1)), full((H, 1)),
        ],
        out_specs=[
            pl.BlockSpec((RB, H), lambda i: (i, 0)),
            pl.BlockSpec((RB, H), lambda i: (i, 0)),
            pl.BlockSpec((RB, 1), lambda i: (i, 0)),
            pl.BlockSpec((RB, 1), lambda i: (i, 0)),
        ],
        out_shape=[
            jax.ShapeDtypeStruct((NP, H), F32),
            jax.ShapeDtypeStruct((NP, H), F32),
            jax.ShapeDtypeStruct((NP, 1), F32),
            jax.ShapeDtypeStruct((NP, 1), F32),
        ],
        compiler_params=pltpu.CompilerParams(
            dimension_semantics=("arbitrary",)),
    )(hacc, s3, x0, gb, wihT, whhT, bih, bhh, awT, asrc_col, adst_col)


def _tc3a_body(hacc_ref, s_ref, x1_ref, ab_ref, wihT_ref, whhT_ref, bih_ref,
               bhh_ref, mwT_ref, msrc_ref,
               xm_ref, ssrcm_ref, x2sum_ref, maxs_ref, acc_ref, mx_ref):
    i = pl.program_id(0)

    @pl.when(i == 0)
    def _():
        acc_ref[...] = jnp.zeros_like(acc_ref)
        mx_ref[0, 0] = -1e30

    hs = hacc_ref[0] + hacc_ref[1]
    ssum = s_ref[0] + s_ref[1]
    h = _elu(hs / (ssum + 1e-16) + ab_ref[...])
    x1 = x1_ref[...]
    x2 = jnp.maximum(
        _gru_block(h, x1, wihT_ref[...], whhT_ref[...], bih_ref[...],
                   bhh_ref[...]), 0.0)
    xm = jnp.dot(x2, mwT_ref[...], preferred_element_type=F32)
    xm_ref[...] = xm
    sm = jnp.dot(xm, msrc_ref[...], preferred_element_type=F32)
    rows = i * RB + lax.broadcasted_iota(jnp.int32, (RB, 1), 0)
    valid = rows < N
    sm = jnp.where(valid, sm, -1e30)
    ssrcm_ref[...] = sm
    acc_ref[...] += jnp.sum(jnp.where(valid, x2, 0.0), axis=0, keepdims=True)
    mx_ref[0, 0] = jnp.maximum(mx_ref[0, 0], jnp.max(sm))
    x2sum_ref[...] = acc_ref[...]
    maxs_ref[0, 0] = mx_ref[0, 0]


def _tc3a(hacc, s3, x1, ab, wihT, whhT, bih, bhh, mwT, msrc_col):
    full = lambda s: pl.BlockSpec(s, lambda i: (0,) * len(s))
    return pl.pallas_call(
        _tc3a_body,
        grid=(NB,),
        in_specs=[
            pl.BlockSpec((2, RB, H), lambda i: (0, i, 0)),
            pl.BlockSpec((2, RB, 1), lambda i: (0, i, 0)),
            pl.BlockSpec((RB, H), lambda i: (i, 0)),
            full((1, H)), full((H, 3 * H)), full((H, 3 * H)),
            full((1, 3 * H)), full((1, 3 * H)), full((H, H)),
            full((H, 1)),
        ],
        out_specs=[
            pl.BlockSpec((RB, H), lambda i: (i, 0)),
            pl.BlockSpec((RB, 1), lambda i: (i, 0)),
            pl.BlockSpec((1, H), lambda i: (0, 0)),
            pl.BlockSpec((1, 1), lambda i: (0, 0),
                         memory_space=pltpu.SMEM),
        ],
        out_shape=[
            jax.ShapeDtypeStruct((NP, H), F32),
            jax.ShapeDtypeStruct((NP, 1), F32),
            jax.ShapeDtypeStruct((1, H), F32),
            jax.ShapeDtypeStruct((1, 1), F32),
        ],
        scratch_shapes=[
            pltpu.VMEM((1, H), F32),
            pltpu.SMEM((1, 1), F32),
        ],
        compiler_params=pltpu.CompilerParams(
            dimension_semantics=("arbitrary",)),
    )(hacc, s3, x1, ab, wihT, whhT, bih, bhh, mwT, msrc_col)


def _tc3b_body(xm_ref, sm_ref, x2sum_ref, maxs_ref, mwT_ref, mdst_ref,
               mb_ref, wihT_ref, whhT_ref, bih_ref, bhh_ref, l2T_ref, l2b_ref,
               out_ref, sw_ref, wxm_ref):
    i = pl.program_id(0)

    @pl.when(i == 0)
    def _():
        sw_ref[0, 0] = 0.0
        wxm_ref[...] = jnp.zeros_like(wxm_ref)

    out0 = jnp.maximum(x2sum_ref[...], 0.0)
    cm = jnp.dot(out0, mwT_ref[...], preferred_element_type=F32)
    c = jnp.sum(cm * mdst_ref[...])
    mx = _lk(maxs_ref[0, 0] + c)
    am = _lk(sm_ref[...] + c)
    w = jnp.exp(am - mx)
    sw_ref[0, 0] += jnp.sum(w)
    wxm_ref[...] += jnp.sum(w * xm_ref[...], axis=0, keepdims=True)
    h3 = _elu(wxm_ref[...] / (sw_ref[0, 0] + 1e-16) + mb_ref[...])
    og = jnp.maximum(
        _gru_block(h3, out0, wihT_ref[...], whhT_ref[...], bih_ref[...],
                   bhh_ref[...]), 0.0)
    out_ref[...] = jnp.dot(og, l2T_ref[...], preferred_element_type=F32) \
        + l2b_ref[...]


def _tc3b(xm, ssrcm, x2sum, maxs, mwT, mdst_row, mb, wihT, whhT, bih, bhh,
          l2T, l2b):
    full = lambda s: pl.BlockSpec(s, lambda i: (0,) * len(s))
    return pl.pallas_call(
        _tc3b_body,
        grid=(NB,),
        in_specs=[
            pl.BlockSpec((RB, H), lambda i: (i, 0)),
            pl.BlockSpec((RB, 1), lambda i: (i, 0)),
            full((1, H)),
            pl.BlockSpec((1, 1), lambda i: (0, 0),
                         memory_space=pltpu.SMEM),
            full((H, H)), full((1, H)), full((1, H)),
            full((H, 3 * H)), full((H, 3 * H)),
            full((1, 3 * H)), full((1, 3 * H)),
            full((H, H)), full((1, H)),
        ],
        out_specs=pl.BlockSpec((1, H), lambda i: (0, 0)),
        out_shape=jax.ShapeDtypeStruct((1, H), F32),
        scratch_shapes=[
            pltpu.SMEM((1, 1), F32),
            pltpu.VMEM((1, H), F32),
        ],
        compiler_params=pltpu.CompilerParams(
            dimension_semantics=("arbitrary",)),
    )(xm, ssrcm, x2sum, maxs, mwT, mdst_row, mb, wihT, whhT, bih, bhh,
      l2T, l2b)


_MESH = dict(core_axis_name="c", subcore_axis_name="s",
             num_cores=2, num_subcores=16)


def _zero_rows(rows_ref, nrows):
    def zrow(e, _):
        for kk in range(H // 16):
            rows_ref[e, pl.ds(kk * 16, 16)] = jnp.zeros((16,), F32)
        return 0
    lax.fori_loop(0, nrows, zrow, 0)


def _prologue(rows_ref, s_part, hacc, s_sp, sid, c):
    _zero_rows(rows_ref, c)
    _zero_rows(s_part, NS)

    def zh(j, _):
        pltpu.sync_copy(rows_ref, hacc.at[pl.ds(sid * NPT + j * c, c)])
        return 0
    lax.fori_loop(0, NPT // c, zh, 0)

    @pl.when(sid < NS // 8)
    def _():
        pltpu.sync_copy(rows_ref.at[pl.ds(0, 8)], s_sp.at[pl.ds(sid * 8, 8)])


def _epilogue(sid, cid, s_part, hacc, s_sp, hacc_out, s_out, c):
    lane = lax.iota(jnp.int32, 16)
    for j in range(NS // 16):
        pltpu.sync_copy(s_part.at[pl.ds(j * 16, 16)], s_sp.at[lane + j * 16],
                        add=True)
    plsc.subcore_barrier()

    def wh(j, _):
        sl = pl.ds(sid * NPT + j * c, c)
        pltpu.sync_copy(hacc.at[sl], hacc_out.at[cid].at[sl])
        return 0
    lax.fori_loop(0, NPT // c, wh, 0)

    @pl.when(sid < NS // 8)
    def _():
        ssl = pl.ds(sid * 8, 8)
        pltpu.sync_copy(s_sp.at[ssl], s_out.at[cid].at[ssl])


def _sc1_body(am_hbm, sr_hbm, w1e_hbm, attl_hbm, src_hbm, dst_hbm,
              ea_hbm, hacc_out, s_out, hacc, s_sp, w1ev, attlv,
              s_part, srcb, dstb, eab, srbuf, amrows, scat,
              tbuf, sem_a, sem_sr, sem_sc):
    cid = lax.axis_index("c")
    sid = lax.axis_index("s")
    wid = cid * 16 + sid
    pltpu.sync_copy(w1e_hbm, w1ev)
    pltpu.sync_copy(attl_hbm, attlv)
    _prologue(scat.at[0], s_part, hacc, s_sp, sid, C1)
    plsc.subcore_barrier()
    w1 = [w1ev[pl.ds(kk * 16, 16)] for kk in range(8)]
    al = [attlv[pl.ds(kk * 16, 16)] for kk in range(8)]
    lane = lax.iota(jnp.int32, 16)

    def load_batch(j, jb):
        row = wid * CPW1 + j * BSZ
        sl = pl.ds(jb * BSZ, BSZ)
        pltpu.sync_copy(src_hbm.at[pl.ds(row, BSZ)], srcb.at[sl])
        pltpu.sync_copy(dst_hbm.at[pl.ds(row, BSZ)], dstb.at[sl])
        pltpu.sync_copy(ea_hbm.at[pl.ds(row, BSZ)], eab.at[sl])

    def issue(kn):
        jbn = (kn // BSZ) & 1
        rown = jbn * BSZ + kn % BSZ
        bn = kn & 1
        pltpu.async_copy(sr_hbm.at[dstb.at[rown]], srbuf.at[bn],
                         sem_sr.at[bn])
        pltpu.async_copy(am_hbm.at[srcb.at[rown]], amrows.at[bn],
                         sem_a.at[bn])

        @pl.when(kn >= 3)
        def _():
            pltpu.make_async_copy(scat.at[kn % 3], hacc.at[dstb.at[0]],
                                  sem_sc.at[kn % 3]).wait()

    load_batch(0, 0)
    issue(0)

    def chunk(k, _):
        b = k & 1
        b3 = k % 3
        row = ((k // BSZ) & 1) * BSZ + k % BSZ
        kn = k + 1

        @pl.when(kn < CPW1)
        def _():
            @pl.when(kn % BSZ == 0)
            def _():
                load_batch(kn // BSZ, (kn // BSZ) & 1)
            issue(kn)

        pltpu.make_async_copy(am_hbm.at[srcb.at[row]], amrows.at[b],
                              sem_a.at[b]).wait()
        pltpu.make_async_copy(sr_hbm.at[dstb.at[row]], srbuf.at[b],
                              sem_sr.at[b]).wait()

        def grp_t(g, _):
            ea16 = eab[row, pl.ds(g * 16, 16)]
            t16 = jnp.zeros((16,), F32)
            for e16 in range(16):
                e = g * 16 + e16
                ea_e = ea16[e16]
                acc = jnp.zeros((16,), F32)
                for kk in range(8):
                    v = amrows[b, e, pl.ds(kk * 16, 16)] + ea_e * w1[kk]
                    v = jnp.maximum(v, 0.01 * v)
                    acc = acc + al[kk] * v
                t16 = jnp.where(lane == e16, jnp.sum(acc), t16)
            dst16 = dstb[row, pl.ds(g * 16, 16)]
            t16 = t16 + srbuf[b, pl.ds(g * 16, 16)]
            t16 = jnp.maximum(t16, 0.01 * t16)
            ee16 = jnp.exp(t16)
            tbuf[pl.ds(g * 16, 16)] = ee16
            plsc.addupdate_scatter(s_part, [dst16 >> 7, dst16 & 127], ee16)
            return 0
        lax.fori_loop(0, C1 // 16, grp_t, 0)

        def grp_s(g, _):
            ee16 = tbuf[pl.ds(g * 16, 16)]
            for e16 in range(16):
                e = g * 16 + e16
                ee_e = ee16[e16]
                for kk in range(8):
                    scat[b3, e, pl.ds(kk * 16, 16)] = \
                        amrows[b, e, pl.ds(H + kk * 16, 16)] * ee_e
            return 0
        lax.fori_loop(0, C1 // 16, grp_s, 0)
        pltpu.async_copy(scat.at[b3], hacc.at[dstb.at[row]], sem_sc.at[b3],
                         add=True)
        return 0
    lax.fori_loop(0, CPW1, chunk, 0)
    for j in range(3):
        pltpu.make_async_copy(scat.at[(CPW1 - 3 + j) % 3],
                              hacc.at[dstb.at[0]],
                              sem_sc.at[(CPW1 - 3 + j) % 3]).wait()
    plsc.subcore_barrier()
    _epilogue(sid, cid, s_part, hacc, s_sp, hacc_out, s_out, C1)


@functools.cache
def _sc1_built():
    return pl.kernel(
        _sc1_body,
        out_type=[
            jax.ShapeDtypeStruct((2, NP, H), F32),
            jax.ShapeDtypeStruct((2, NS, 128), F32),
        ],
        mesh=plsc.VectorSubcoreMesh(**_MESH),
        scratch_types=[
            pltpu.VMEM_SHARED((NP, H), F32),
            pltpu.VMEM_SHARED((NS, 128), F32),
            pltpu.VMEM((H,), F32),
            pltpu.VMEM((H,), F32),
            pltpu.VMEM((NS, 128), F32),
            pltpu.VMEM((2 * BSZ, C1), jnp.int32),
            pltpu.VMEM((2 * BSZ, C1), jnp.int32),
            pltpu.VMEM((2 * BSZ, C1), F32),
            pltpu.VMEM((2, C1), F32),
            pltpu.VMEM((2, C1, 2 * H), F32),
            pltpu.VMEM((3, C1, H), F32),
            pltpu.VMEM((C1,), F32),
            pltpu.SemaphoreType.DMA((2,)),
            pltpu.SemaphoreType.DMA((2,)),
            pltpu.SemaphoreType.DMA((3,)),
        ],
        compiler_params=pltpu.CompilerParams(needs_layout_passes=False),
    )


def _sc1(*args):
    return _sc1_built()(*args)


def _sc2_body(xs_hbm, ssrc_hbm, sdst_hbm, src_hbm, dst_hbm, hacc_out, s_out,
              hacc, s_sp, s_part, srcb, dstb, sabuf, sbbuf,
              xsrows, ebuf, sem_x, sem_a, sem_b, sem_sc):
    cid = lax.axis_index("c")
    sid = lax.axis_index("s")
    wid = cid * 16 + sid
    _prologue(xsrows.at[0], s_part, hacc, s_sp, sid, C2)
    plsc.subcore_barrier()

    def load_batch(j, jb):
        row = wid * CPW2 + j * BSZ
        sl = pl.ds(jb * BSZ, BSZ)
        pltpu.sync_copy(src_hbm.at[pl.ds(row, BSZ)], srcb.at[sl])
        pltpu.sync_copy(dst_hbm.at[pl.ds(row, BSZ)], dstb.at[sl])

    def issue(kn):
        rown = ((kn // BSZ) & 1) * BSZ + kn % BSZ
        bn = kn & 1
        bn3 = kn % 3
        pltpu.async_copy(ssrc_hbm.at[srcb.at[rown]], sabuf.at[bn],
                         sem_a.at[bn])
        pltpu.async_copy(sdst_hbm.at[dstb.at[rown]], sbbuf.at[bn],
                         sem_b.at[bn])

        @pl.when(kn >= 3)
        def _():
            pltpu.make_async_copy(xsrows.at[bn3], hacc.at[dstb.at[0]],
                                  sem_sc.at[bn3]).wait()
        pltpu.async_copy(xs_hbm.at[srcb.at[rown]], xsrows.at[bn3],
                         sem_x.at[bn3])

    load_batch(0, 0)
    issue(0)

    def chunk(k, _):
        b = k & 1
        b3 = k % 3
        row = ((k // BSZ) & 1) * BSZ + k % BSZ
        kn = k + 1

        @pl.when(kn < CPW2)
        def _():
            @pl.when(kn % BSZ == 0)
            def _():
                load_batch(kn // BSZ, (kn // BSZ) & 1)
            issue(kn)

        pltpu.make_async_copy(ssrc_hbm.at[srcb.at[row]], sabuf.at[b],
                              sem_a.at[b]).wait()
        pltpu.make_async_copy(sdst_hbm.at[dstb.at[row]], sbbuf.at[b],
                              sem_b.at[b]).wait()
        for g in range(C2 // 16):
            dst16 = dstb[row, pl.ds(g * 16, 16)]
            a16 = sabuf[b, pl.ds(g * 16, 16)] + sbbuf[b, pl.ds(g * 16, 16)]
            a16 = jnp.maximum(a16, 0.01 * a16)
            ee16 = jnp.exp(a16)
            ebuf[pl.ds(g * 16, 16)] = ee16
            plsc.addupdate_scatter(s_part, [dst16 >> 7, dst16 & 127], ee16)
        pltpu.make_async_copy(xs_hbm.at[srcb.at[row]], xsrows.at[b3],
                              sem_x.at[b3]).wait()

        def grp_s(g, _):
            ee16 = ebuf[pl.ds(g * 16, 16)]
            for e16 in range(16):
                e = g * 16 + e16
                ee_e = ee16[e16]
                for kk in range(8):
                    xsrows[b3, e, pl.ds(kk * 16, 16)] = \
                        xsrows[b3, e, pl.ds(kk * 16, 16)] * ee_e
            return 0
        lax.fori_loop(0, C2 // 16, grp_s, 0)
        pltpu.async_copy(xsrows.at[b3], hacc.at[dstb.at[row]], sem_sc.at[b3],
                         add=True)
        return 0
    lax.fori_loop(0, CPW2, chunk, 0)
    for j in range(3):
        pltpu.make_async_copy(xsrows.at[(CPW2 - 3 + j) % 3],
                              hacc.at[dstb.at[0]],
                              sem_sc.at[(CPW2 - 3 + j) % 3]).wait()
    plsc.subcore_barrier()
    _epilogue(sid, cid, s_part, hacc, s_sp, hacc_out, s_out, C2)


@functools.cache
def _sc2_built():
    return pl.kernel(
        _sc2_body,
        out_type=[
            jax.ShapeDtypeStruct((2, NP, H), F32),
            jax.ShapeDtypeStruct((2, NS, 128), F32),
        ],
        mesh=plsc.VectorSubcoreMesh(**_MESH),
        scratch_types=[
            pltpu.VMEM_SHARED((NP, H), F32),
            pltpu.VMEM_SHARED((NS, 128), F32),
            pltpu.VMEM((NS, 128), F32),
            pltpu.VMEM((2 * BSZ, C2), jnp.int32),
            pltpu.VMEM((2 * BSZ, C2), jnp.int32),
            pltpu.VMEM((2, C2), F32),
            pltpu.VMEM((2, C2), F32),
            pltpu.VMEM((3, C2, H), F32),
            pltpu.VMEM((C2,), F32),
            pltpu.SemaphoreType.DMA((3,)),
            pltpu.SemaphoreType.DMA((2,)),
            pltpu.SemaphoreType.DMA((2,)),
            pltpu.SemaphoreType.DMA((3,)),
        ],
        compiler_params=pltpu.CompilerParams(needs_layout_passes=False),
    )


def _sc2(*args):
    return _sc2_built()(*args)


def kernel(node_attr, edge_index, edge_attr, params):
    p = params
    na = jnp.pad(node_attr, ((0, NP - N), (0, 0)))
    src_p = jnp.pad(edge_index[0], (0, EP - E), constant_values=NP - 1)
    dst_p = jnp.pad(edge_index[1], (0, EP - E), constant_values=NP - 1)
    ea_p = jnp.pad(edge_attr[:, 0], (0, EP - E))
    src2d1 = src_p.reshape(EP // C1, C1)
    dst2d1 = dst_p.reshape(EP // C1, C1)
    ea2d1 = ea_p.reshape(EP // C1, C1)
    src2d2 = src_p.reshape(EP // C2, C2)
    dst2d2 = dst_p.reshape(EP // C2, C2)

    l1w = p["lin1_w"].T
    l1b = p["lin1_b"].reshape(1, H)
    w1aT = p["gate_lin1_w"][:, :H].T
    w1e = p["gate_lin1_w"][:, H]
    g2T = p["gate_lin2_w"].T
    attr_col = p["gate_att_r"].reshape(H, 1)

    x0, am_mat, sr = _tc1(na, l1w, l1b, w1aT, g2T, attr_col)
    hacc, s1 = _sc1(am_mat, sr.reshape(NP), w1e, p["gate_att_l"],
                    src2d1, dst2d1, ea2d1)
    x1, xs, ssrc, sdst = _tc2(
        hacc, s1.reshape(2, NP, 1), x0, p["gate_bias"].reshape(1, H),
        p["gru1_wih"].T, p["gru1_whh"].T,
        p["gru1_bih"].reshape(1, 3 * H), p["gru1_bhh"].reshape(1, 3 * H),
        p["atom_w"].T, p["atom_att_src"].reshape(H, 1),
        p["atom_att_dst"].reshape(H, 1))
    hacc2, s2 = _sc2(xs, ssrc.reshape(NP), sdst.reshape(NP), src2d2, dst2d2)
    xm, ssrcm, x2sum, maxs = _tc3a(
        hacc2, s2.reshape(2, NP, 1), x1, p["atom_bias"].reshape(1, H),
        p["gru2_wih"].T, p["gru2_whh"].T,
        p["gru2_bih"].reshape(1, 3 * H), p["gru2_bhh"].reshape(1, 3 * H),
        p["mol_w"].T, p["mol_att_src"].reshape(H, 1))
    out = _tc3b(
        xm, ssrcm, x2sum, maxs, p["mol_w"].T,
        p["mol_att_dst"].reshape(1, H), p["mol_bias"].reshape(1, H),
        p["grum_wih"].T, p["grum_whh"].T,
        p["grum_bih"].reshape(1, 3 * H), p["grum_bhh"].reshape(1, 3 * H),
        p["lin2_w"].T, p["lin2_b"].reshape(1, H))
    return out


# speedup vs baseline: 10.9683x; 1.0787x over previous
import functools

import jax
import jax.numpy as jnp
from jax import lax
from jax.experimental import pallas as pl
from jax.experimental.pallas import tpu as pltpu
from jax.experimental.pallas import tpu_sc as plsc

N = 10000
E = 160000
H = 128
NP = 10240
EP = 163840
C1 = 32
C2 = 64
BSZ = 8
NWORK = 32
CPW1 = EP // C1 // NWORK
CPW2 = EP // C2 // NWORK
RB = 512
NB = NP // RB
NPT = NP // 16
NS = NP // 128
F32 = jnp.float32


def _lk(x):
    return jnp.where(x >= 0, x, 0.01 * x)


def _elu(x):
    return jnp.where(x > 0, x, jnp.exp(x) - 1.0)


def _gru_block(h, hid, wihT, whhT, bih, bhh):
    gi = jnp.dot(h, wihT, preferred_element_type=F32) + bih
    gh = jnp.dot(hid, whhT, preferred_element_type=F32) + bhh
    r = jax.nn.sigmoid(gi[:, :H] + gh[:, :H])
    z = jax.nn.sigmoid(gi[:, H:2 * H] + gh[:, H:2 * H])
    nn_ = jnp.tanh(gi[:, 2 * H:] + r * gh[:, 2 * H:])
    return (1.0 - z) * nn_ + z * hid


def _tc1_body(na_ref, l1w_ref, l1b_ref, w1aT_ref, g2T_ref, attr_ref,
              x0_ref, am_ref, sr_ref):
    x0 = _lk(na_ref[...] * l1w_ref[...] + l1b_ref[...])
    x0_ref[...] = x0
    am_ref[:, :H] = jnp.dot(x0, w1aT_ref[...], preferred_element_type=F32)
    am_ref[:, H:] = jnp.dot(x0, g2T_ref[...], preferred_element_type=F32)
    sr_ref[...] = jnp.dot(x0, attr_ref[...], preferred_element_type=F32)


def _tc1(na, l1w, l1b, w1aT, g2T, attr_col):
    full = lambda s: pl.BlockSpec(s, lambda i: (0,) * len(s))
    return pl.pallas_call(
        _tc1_body,
        grid=(NB,),
        in_specs=[
            pl.BlockSpec((RB, 1), lambda i: (i, 0)),
            full((1, H)), full((1, H)), full((H, H)), full((H, H)),
            full((H, 1)),
        ],
        out_specs=[
            pl.BlockSpec((RB, H), lambda i: (i, 0)),
            pl.BlockSpec((RB, 2 * H), lambda i: (i, 0)),
            pl.BlockSpec((RB, 1), lambda i: (i, 0)),
        ],
        out_shape=[
            jax.ShapeDtypeStruct((NP, H), F32),
            jax.ShapeDtypeStruct((NP, 2 * H), F32),
            jax.ShapeDtypeStruct((NP, 1), F32),
        ],
        compiler_params=pltpu.CompilerParams(
            dimension_semantics=("arbitrary",)),
    )(na, l1w, l1b, w1aT, g2T, attr_col)


def _tc2_body(hacc_ref, s_ref, x0_ref, gb_ref, wihT_ref, whhT_ref, bih_ref,
              bhh_ref, awT_ref, asrc_ref, adst_ref,
              x1_ref, xs_ref, ssrc_ref, sdst_ref):
    hs = hacc_ref[0] + hacc_ref[1]
    ssum = s_ref[0] + s_ref[1]
    h = _elu(hs / (ssum + 1e-16) + gb_ref[...])
    x0 = x0_ref[...]
    x1 = jnp.maximum(
        _gru_block(h, x0, wihT_ref[...], whhT_ref[...], bih_ref[...],
                   bhh_ref[...]), 0.0)
    x1_ref[...] = x1
    xs = jnp.dot(x1, awT_ref[...], preferred_element_type=F32)
    xs_ref[...] = xs
    ssrc_ref[...] = jnp.dot(xs, asrc_ref[...], preferred_element_type=F32)
    sdst_ref[...] = jnp.dot(xs, adst_ref[...], preferred_element_type=F32)


def _tc2(hacc, s3, x0, gb, wihT, whhT, bih, bhh, awT, asrc_col, adst_col):
    full = lambda s: pl.BlockSpec(s, lambda i: (0,) * len(s))
    return pl.pallas_call(
        _tc2_body,
        grid=(NB,),
        in_specs=[
            pl.BlockSpec((2, RB, H), lambda i: (0, i, 0)),
            pl.BlockSpec((2, RB, 1), lambda i: (0, i, 0)),
            pl.BlockSpec((RB, H), lambda i: (i, 0)),
            full((1, H)), full((H, 3 * H)), full((H, 3 * H)),
            full((1, 3 * H)), full((1, 3 * H)), full((H, H)),
            full((H, 1)), full((H, 1)),
        ],
        out_specs=[
            pl.BlockSpec((RB, H), lambda i: (i, 0)),
            pl.BlockSpec((RB, H), lambda i: (i, 0)),
            pl.BlockSpec((RB, 1), lambda i: (i, 0)),
            pl.BlockSpec((RB, 1), lambda i: (i, 0)),
        ],
        out_shape=[
            jax.ShapeDtypeStruct((NP, H), F32),
            jax.ShapeDtypeStruct((NP, H), F32),
            jax.ShapeDtypeStruct((NP, 1), F32),
            jax.ShapeDtypeStruct((NP, 1), F32),
        ],
        compiler_params=pltpu.CompilerParams(
            dimension_semantics=("arbitrary",)),
    )(hacc, s3, x0, gb, wihT, whhT, bih, bhh, awT, asrc_col, adst_col)


def _tc3a_body(hacc_ref, s_ref, x1_ref, ab_ref, wihT_ref, whhT_ref, bih_ref,
               bhh_ref, mwT_ref, msrc_ref,
               xm_ref, ssrcm_ref, x2sum_ref, maxs_ref, acc_ref, mx_ref):
    i = pl.program_id(0)

    @pl.when(i == 0)
    def _():
        acc_ref[...] = jnp.zeros_like(acc_ref)
        mx_ref[0, 0] = -1e30

    hs = hacc_ref[0] + hacc_ref[1]
    ssum = s_ref[0] + s_ref[1]
    h = _elu(hs / (ssum + 1e-16) + ab_ref[...])
    x1 = x1_ref[...]
    x2 = jnp.maximum(
        _gru_block(h, x1, wihT_ref[...], whhT_ref[...], bih_ref[...],
                   bhh_ref[...]), 0.0)
    xm = jnp.dot(x2, mwT_ref[...], preferred_element_type=F32)
    xm_ref[...] = xm
    sm = jnp.dot(xm, msrc_ref[...], preferred_element_type=F32)
    rows = i * RB + lax.broadcasted_iota(jnp.int32, (RB, 1), 0)
    valid = rows < N
    sm = jnp.where(valid, sm, -1e30)
    ssrcm_ref[...] = sm
    acc_ref[...] += jnp.sum(jnp.where(valid, x2, 0.0), axis=0, keepdims=True)
    mx_ref[0, 0] = jnp.maximum(mx_ref[0, 0], jnp.max(sm))
    x2sum_ref[...] = acc_ref[...]
    maxs_ref[0, 0] = mx_ref[0, 0]


def _tc3a(hacc, s3, x1, ab, wihT, whhT, bih, bhh, mwT, msrc_col):
    full = lambda s: pl.BlockSpec(s, lambda i: (0,) * len(s))
    return pl.pallas_call(
        _tc3a_body,
        grid=(NB,),
        in_specs=[
            pl.BlockSpec((2, RB, H), lambda i: (0, i, 0)),
            pl.BlockSpec((2, RB, 1), lambda i: (0, i, 0)),
            pl.BlockSpec((RB, H), lambda i: (i, 0)),
            full((1, H)), full((H, 3 * H)), full((H, 3 * H)),
            full((1, 3 * H)), full((1, 3 * H)), full((H, H)),
            full((H, 1)),
        ],
        out_specs=[
            pl.BlockSpec((RB, H), lambda i: (i, 0)),
            pl.BlockSpec((RB, 1), lambda i: (i, 0)),
            pl.BlockSpec((1, H), lambda i: (0, 0)),
            pl.BlockSpec((1, 1), lambda i: (0, 0),
                         memory_space=pltpu.SMEM),
        ],
        out_shape=[
            jax.ShapeDtypeStruct((NP, H), F32),
            jax.ShapeDtypeStruct((NP, 1), F32),
            jax.ShapeDtypeStruct((1, H), F32),
            jax.ShapeDtypeStruct((1, 1), F32),
        ],
        scratch_shapes=[
            pltpu.VMEM((1, H), F32),
            pltpu.SMEM((1, 1), F32),
        ],
        compiler_params=pltpu.CompilerParams(
            dimension_semantics=("arbitrary",)),
    )(hacc, s3, x1, ab, wihT, whhT, bih, bhh, mwT, msrc_col)


def _tc3b_body(xm_ref, sm_ref, x2sum_ref, maxs_ref, mwT_ref, mdst_ref,
               mb_ref, wihT_ref, whhT_ref, bih_ref, bhh_ref, l2T_ref, l2b_ref,
               out_ref, sw_ref, wxm_ref):
    i = pl.program_id(0)

    @pl.when(i == 0)
    def _():
        sw_ref[0, 0] = 0.0
        wxm_ref[...] = jnp.zeros_like(wxm_ref)

    out0 = jnp.maximum(x2sum_ref[...], 0.0)
    cm = jnp.dot(out0, mwT_ref[...], preferred_element_type=F32)
    c = jnp.sum(cm * mdst_ref[...])
    mx = _lk(maxs_ref[0, 0] + c)
    am = _lk(sm_ref[...] + c)
    w = jnp.exp(am - mx)
    sw_ref[0, 0] += jnp.sum(w)
    wxm_ref[...] += jnp.sum(w * xm_ref[...], axis=0, keepdims=True)
    h3 = _elu(wxm_ref[...] / (sw_ref[0, 0] + 1e-16) + mb_ref[...])
    og = jnp.maximum(
        _gru_block(h3, out0, wihT_ref[...], whhT_ref[...], bih_ref[...],
                   bhh_ref[...]), 0.0)
    out_ref[...] = jnp.dot(og, l2T_ref[...], preferred_element_type=F32) \
        + l2b_ref[...]


def _tc3b(xm, ssrcm, x2sum, maxs, mwT, mdst_row, mb, wihT, whhT, bih, bhh,
          l2T, l2b):
    full = lambda s: pl.BlockSpec(s, lambda i: (0,) * len(s))
    return pl.pallas_call(
        _tc3b_body,
        grid=(NB,),
        in_specs=[
            pl.BlockSpec((RB, H), lambda i: (i, 0)),
            pl.BlockSpec((RB, 1), lambda i: (i, 0)),
            full((1, H)),
            pl.BlockSpec((1, 1), lambda i: (0, 0),
                         memory_space=pltpu.SMEM),
            full((H, H)), full((1, H)), full((1, H)),
            full((H, 3 * H)), full((H, 3 * H)),
            full((1, 3 * H)), full((1, 3 * H)),
            full((H, H)), full((1, H)),
        ],
        out_specs=pl.BlockSpec((1, H), lambda i: (0, 0)),
        out_shape=jax.ShapeDtypeStruct((1, H), F32),
        scratch_shapes=[
            pltpu.SMEM((1, 1), F32),
            pltpu.VMEM((1, H), F32),
        ],
        compiler_params=pltpu.CompilerParams(
            dimension_semantics=("arbitrary",)),
    )(xm, ssrcm, x2sum, maxs, mwT, mdst_row, mb, wihT, whhT, bih, bhh,
      l2T, l2b)


_MESH = dict(core_axis_name="c", subcore_axis_name="s",
             num_cores=2, num_subcores=16)


def _zero_rows(rows_ref, nrows):
    def zrow(e, _):
        for kk in range(H // 16):
            rows_ref[e, pl.ds(kk * 16, 16)] = jnp.zeros((16,), F32)
        return 0
    lax.fori_loop(0, nrows, zrow, 0)


def _prologue(rows_ref, s_part, hacc, s_sp, sid, c):
    _zero_rows(rows_ref, c)
    _zero_rows(s_part, NS)

    def zh(j, _):
        pltpu.sync_copy(rows_ref, hacc.at[pl.ds(sid * NPT + j * c, c)])
        return 0
    lax.fori_loop(0, NPT // c, zh, 0)

    @pl.when(sid < NS // 8)
    def _():
        pltpu.sync_copy(rows_ref.at[pl.ds(0, 8)], s_sp.at[pl.ds(sid * 8, 8)])


def _epilogue(sid, cid, s_part, hacc, s_sp, hacc_out, s_out, c):
    lane = lax.iota(jnp.int32, 16)
    for j in range(NS // 16):
        pltpu.sync_copy(s_part.at[pl.ds(j * 16, 16)], s_sp.at[lane + j * 16],
                        add=True)
    plsc.subcore_barrier()

    def wh(j, _):
        sl = pl.ds(sid * NPT + j * c, c)
        pltpu.sync_copy(hacc.at[sl], hacc_out.at[cid].at[sl])
        return 0
    lax.fori_loop(0, NPT // c, wh, 0)

    @pl.when(sid < NS // 8)
    def _():
        ssl = pl.ds(sid * 8, 8)
        pltpu.sync_copy(s_sp.at[ssl], s_out.at[cid].at[ssl])


def _sc1_body(am_hbm, sr_hbm, w1e_hbm, attl_hbm, src_hbm, dst_hbm,
              ea_hbm, hacc_out, s_out, hacc, s_sp, w1ev, attlv,
              s_part, srcb, dstb, eab, srbuf, amrows, scat,
              tbuf, sem_a, sem_sr, sem_sc):
    cid = lax.axis_index("c")
    sid = lax.axis_index("s")
    wid = cid * 16 + sid
    pltpu.sync_copy(w1e_hbm, w1ev)
    pltpu.sync_copy(attl_hbm, attlv)
    _prologue(scat.at[0], s_part, hacc, s_sp, sid, C1)
    plsc.subcore_barrier()
    w1 = [w1ev[pl.ds(kk * 16, 16)] for kk in range(8)]
    al = [attlv[pl.ds(kk * 16, 16)] for kk in range(8)]
    lane = lax.iota(jnp.int32, 16)

    def load_batch(j, jb):
        row = wid * CPW1 + j * BSZ
        sl = pl.ds(jb * BSZ, BSZ)
        pltpu.sync_copy(src_hbm.at[pl.ds(row, BSZ)], srcb.at[sl])
        pltpu.sync_copy(dst_hbm.at[pl.ds(row, BSZ)], dstb.at[sl])
        pltpu.sync_copy(ea_hbm.at[pl.ds(row, BSZ)], eab.at[sl])

    def issue(kn):
        jbn = (kn // BSZ) & 1
        rown = jbn * BSZ + kn % BSZ
        bn = kn & 1
        pltpu.async_copy(sr_hbm.at[dstb.at[rown]], srbuf.at[bn],
                         sem_sr.at[bn])
        pltpu.async_copy(am_hbm.at[srcb.at[rown]], amrows.at[bn],
                         sem_a.at[bn])

        @pl.when(kn >= 3)
        def _():
            pltpu.make_async_copy(scat.at[kn % 3], hacc.at[dstb.at[0]],
                                  sem_sc.at[kn % 3]).wait()

    load_batch(0, 0)
    issue(0)

    def chunk(k, _):
        b = k & 1
        b3 = k % 3
        row = ((k // BSZ) & 1) * BSZ + k % BSZ
        kn = k + 1

        @pl.when(kn < CPW1)
        def _():
            @pl.when(kn % BSZ == 0)
            def _():
                load_batch(kn // BSZ, (kn // BSZ) & 1)
            issue(kn)

        pltpu.make_async_copy(am_hbm.at[srcb.at[row]], amrows.at[b],
                              sem_a.at[b]).wait()
        pltpu.make_async_copy(sr_hbm.at[dstb.at[row]], srbuf.at[b],
                              sem_sr.at[b]).wait()

        @plsc.parallel_loop(0, C1 // 16, 1, unroll=2)
        def grp_t(g):
            ea16 = eab[row, pl.ds(g * 16, 16)]
            t16 = jnp.zeros((16,), F32)
            for e16 in range(16):
                e = g * 16 + e16
                ea_e = ea16[e16]
                acc = jnp.zeros((16,), F32)
                for kk in range(8):
                    v = amrows[b, e, pl.ds(kk * 16, 16)] + ea_e * w1[kk]
                    v = jnp.maximum(v, 0.01 * v)
                    acc = acc + al[kk] * v
                t16 = jnp.where(lane == e16, jnp.sum(acc), t16)
            dst16 = dstb[row, pl.ds(g * 16, 16)]
            t16 = t16 + srbuf[b, pl.ds(g * 16, 16)]
            t16 = jnp.maximum(t16, 0.01 * t16)
            ee16 = jnp.exp(t16)
            tbuf[pl.ds(g * 16, 16)] = ee16
            plsc.addupdate_scatter(s_part, [dst16 >> 7, dst16 & 127], ee16)

        @plsc.parallel_loop(0, C1 // 16, 1, unroll=2)
        def grp_s(g):
            ee16 = tbuf[pl.ds(g * 16, 16)]
            for e16 in range(16):
                e = g * 16 + e16
                ee_e = ee16[e16]
                for kk in range(8):
                    scat[b3, e, pl.ds(kk * 16, 16)] = \
                        amrows[b, e, pl.ds(H + kk * 16, 16)] * ee_e
        pltpu.async_copy(scat.at[b3], hacc.at[dstb.at[row]], sem_sc.at[b3],
                         add=True)
        return 0
    lax.fori_loop(0, CPW1, chunk, 0)
    for j in range(3):
        pltpu.make_async_copy(scat.at[(CPW1 - 3 + j) % 3],
                              hacc.at[dstb.at[0]],
                              sem_sc.at[(CPW1 - 3 + j) % 3]).wait()
    plsc.subcore_barrier()
    _epilogue(sid, cid, s_part, hacc, s_sp, hacc_out, s_out, C1)


@functools.cache
def _sc1_built():
    return pl.kernel(
        _sc1_body,
        out_type=[
            jax.ShapeDtypeStruct((2, NP, H), F32),
            jax.ShapeDtypeStruct((2, NS, 128), F32),
        ],
        mesh=plsc.VectorSubcoreMesh(**_MESH),
        scratch_types=[
            pltpu.VMEM_SHARED((NP, H), F32),
            pltpu.VMEM_SHARED((NS, 128), F32),
            pltpu.VMEM((H,), F32),
            pltpu.VMEM((H,), F32),
            pltpu.VMEM((NS, 128), F32),
            pltpu.VMEM((2 * BSZ, C1), jnp.int32),
            pltpu.VMEM((2 * BSZ, C1), jnp.int32),
            pltpu.VMEM((2 * BSZ, C1), F32),
            pltpu.VMEM((2, C1), F32),
            pltpu.VMEM((2, C1, 2 * H), F32),
            pltpu.VMEM((3, C1, H), F32),
            pltpu.VMEM((C1,), F32),
            pltpu.SemaphoreType.DMA((2,)),
            pltpu.SemaphoreType.DMA((2,)),
            pltpu.SemaphoreType.DMA((3,)),
        ],
        compiler_params=pltpu.CompilerParams(needs_layout_passes=False),
    )


def _sc1(*args):
    return _sc1_built()(*args)


def _sc2_body(xs_hbm, ssrc_hbm, sdst_hbm, src_hbm, dst_hbm, hacc_out, s_out,
              hacc, s_sp, s_part, srcb, dstb, sabuf, sbbuf,
              xsrows, ebuf, sem_x, sem_a, sem_b, sem_sc):
    cid = lax.axis_index("c")
    sid = lax.axis_index("s")
    wid = cid * 16 + sid
    _prologue(xsrows.at[0], s_part, hacc, s_sp, sid, C2)
    plsc.subcore_barrier()

    def load_batch(j, jb):
        row = wid * CPW2 + j * BSZ
        sl = pl.ds(jb * BSZ, BSZ)
        pltpu.sync_copy(src_hbm.at[pl.ds(row, BSZ)], srcb.at[sl])
        pltpu.sync_copy(dst_hbm.at[pl.ds(row, BSZ)], dstb.at[sl])

    def issue(kn):
        rown = ((kn // BSZ) & 1) * BSZ + kn % BSZ
        bn = kn & 1
        bn3 = kn % 3
        pltpu.async_copy(ssrc_hbm.at[srcb.at[rown]], sabuf.at[bn],
                         sem_a.at[bn])
        pltpu.async_copy(sdst_hbm.at[dstb.at[rown]], sbbuf.at[bn],
                         sem_b.at[bn])

        @pl.when(kn >= 3)
        def _():
            pltpu.make_async_copy(xsrows.at[bn3], hacc.at[dstb.at[0]],
                                  sem_sc.at[bn3]).wait()
        pltpu.async_copy(xs_hbm.at[srcb.at[rown]], xsrows.at[bn3],
                         sem_x.at[bn3])

    load_batch(0, 0)
    issue(0)

    def chunk(k, _):
        b = k & 1
        b3 = k % 3
        row = ((k // BSZ) & 1) * BSZ + k % BSZ
        kn = k + 1

        @pl.when(kn < CPW2)
        def _():
            @pl.when(kn % BSZ == 0)
            def _():
                load_batch(kn // BSZ, (kn // BSZ) & 1)
            issue(kn)

        pltpu.make_async_copy(ssrc_hbm.at[srcb.at[row]], sabuf.at[b],
                              sem_a.at[b]).wait()
        pltpu.make_async_copy(sdst_hbm.at[dstb.at[row]], sbbuf.at[b],
                              sem_b.at[b]).wait()
        for g in range(C2 // 16):
            dst16 = dstb[row, pl.ds(g * 16, 16)]
            a16 = sabuf[b, pl.ds(g * 16, 16)] + sbbuf[b, pl.ds(g * 16, 16)]
            a16 = jnp.maximum(a16, 0.01 * a16)
            ee16 = jnp.exp(a16)
            ebuf[pl.ds(g * 16, 16)] = ee16
            plsc.addupdate_scatter(s_part, [dst16 >> 7, dst16 & 127], ee16)
        pltpu.make_async_copy(xs_hbm.at[srcb.at[row]], xsrows.at[b3],
                              sem_x.at[b3]).wait()

        @plsc.parallel_loop(0, C2 // 16, 1, unroll=2)
        def grp_s(g):
            ee16 = ebuf[pl.ds(g * 16, 16)]
            for e16 in range(16):
                e = g * 16 + e16
                ee_e = ee16[e16]
                for kk in range(8):
                    xsrows[b3, e, pl.ds(kk * 16, 16)] = \
                        xsrows[b3, e, pl.ds(kk * 16, 16)] * ee_e
        pltpu.async_copy(xsrows.at[b3], hacc.at[dstb.at[row]], sem_sc.at[b3],
                         add=True)
        return 0
    lax.fori_loop(0, CPW2, chunk, 0)
    for j in range(3):
        pltpu.make_async_copy(xsrows.at[(CPW2 - 3 + j) % 3],
                              hacc.at[dstb.at[0]],
                              sem_sc.at[(CPW2 - 3 + j) % 3]).wait()
    plsc.subcore_barrier()
    _epilogue(sid, cid, s_part, hacc, s_sp, hacc_out, s_out, C2)


@functools.cache
def _sc2_built():
    return pl.kernel(
        _sc2_body,
        out_type=[
            jax.ShapeDtypeStruct((2, NP, H), F32),
            jax.ShapeDtypeStruct((2, NS, 128), F32),
        ],
        mesh=plsc.VectorSubcoreMesh(**_MESH),
        scratch_types=[
            pltpu.VMEM_SHARED((NP, H), F32),
            pltpu.VMEM_SHARED((NS, 128), F32),
            pltpu.VMEM((NS, 128), F32),
            pltpu.VMEM((2 * BSZ, C2), jnp.int32),
            pltpu.VMEM((2 * BSZ, C2), jnp.int32),
            pltpu.VMEM((2, C2), F32),
            pltpu.VMEM((2, C2), F32),
            pltpu.VMEM((3, C2, H), F32),
            pltpu.VMEM((C2,), F32),
            pltpu.SemaphoreType.DMA((3,)),
            pltpu.SemaphoreType.DMA((2,)),
            pltpu.SemaphoreType.DMA((2,)),
            pltpu.SemaphoreType.DMA((3,)),
        ],
        compiler_params=pltpu.CompilerParams(needs_layout_passes=False),
    )


def _sc2(*args):
    return _sc2_built()(*args)


def kernel(node_attr, edge_index, edge_attr, params):
    p = params
    na = jnp.pad(node_attr, ((0, NP - N), (0, 0)))
    src_p = jnp.pad(edge_index[0], (0, EP - E), constant_values=NP - 1)
    dst_p = jnp.pad(edge_index[1], (0, EP - E), constant_values=NP - 1)
    ea_p = jnp.pad(edge_attr[:, 0], (0, EP - E))
    src2d1 = src_p.reshape(EP // C1, C1)
    dst2d1 = dst_p.reshape(EP // C1, C1)
    ea2d1 = ea_p.reshape(EP // C1, C1)
    src2d2 = src_p.reshape(EP // C2, C2)
    dst2d2 = dst_p.reshape(EP // C2, C2)

    l1w = p["lin1_w"].T
    l1b = p["lin1_b"].reshape(1, H)
    w1aT = p["gate_lin1_w"][:, :H].T
    w1e = p["gate_lin1_w"][:, H]
    g2T = p["gate_lin2_w"].T
    attr_col = p["gate_att_r"].reshape(H, 1)

    x0, am_mat, sr = _tc1(na, l1w, l1b, w1aT, g2T, attr_col)
    hacc, s1 = _sc1(am_mat, sr.reshape(NP), w1e, p["gate_att_l"],
                    src2d1, dst2d1, ea2d1)
    x1, xs, ssrc, sdst = _tc2(
        hacc, s1.reshape(2, NP, 1), x0, p["gate_bias"].reshape(1, H),
        p["gru1_wih"].T, p["gru1_whh"].T,
        p["gru1_bih"].reshape(1, 3 * H), p["gru1_bhh"].reshape(1, 3 * H),
        p["atom_w"].T, p["atom_att_src"].reshape(H, 1),
        p["atom_att_dst"].reshape(H, 1))
    hacc2, s2 = _sc2(xs, ssrc.reshape(NP), sdst.reshape(NP), src2d2, dst2d2)
    xm, ssrcm, x2sum, maxs = _tc3a(
        hacc2, s2.reshape(2, NP, 1), x1, p["atom_bias"].reshape(1, H),
        p["gru2_wih"].T, p["gru2_whh"].T,
        p["gru2_bih"].reshape(1, 3 * H), p["gru2_bhh"].reshape(1, 3 * H),
        p["mol_w"].T, p["mol_att_src"].reshape(H, 1))
    out = _tc3b(
        xm, ssrcm, x2sum, maxs, p["mol_w"].T,
        p["mol_att_dst"].reshape(1, H), p["mol_bias"].reshape(1, H),
        p["grum_wih"].T, p["grum_whh"].T,
        p["grum_bih"].reshape(1, 3 * H), p["grum_bhh"].reshape(1, 3 * H),
        p["lin2_w"].T, p["lin2_b"].reshape(1, H))
    return out


# speedup vs baseline: 11.1947x; 1.0206x over previous
import functools

import jax
import jax.numpy as jnp
from jax import lax
from jax.experimental import pallas as pl
from jax.experimental.pallas import tpu as pltpu
from jax.experimental.pallas import tpu_sc as plsc

N = 10000
E = 160000
H = 128
NP = 10240
EP = 163840
C1 = 32
C2 = 64
BSZ = 8
NWORK = 32
CPW1 = EP // C1 // NWORK
CPW2 = EP // C2 // NWORK
RB = 512
NB = NP // RB
NPT = NP // 16
NS = NP // 128
F32 = jnp.float32


def _lk(x):
    return jnp.where(x >= 0, x, 0.01 * x)


def _elu(x):
    return jnp.where(x > 0, x, jnp.exp(x) - 1.0)


def _gru_block(h, hid, wihT, whhT, bih, bhh):
    gi = jnp.dot(h, wihT, preferred_element_type=F32) + bih
    gh = jnp.dot(hid, whhT, preferred_element_type=F32) + bhh
    r = jax.nn.sigmoid(gi[:, :H] + gh[:, :H])
    z = jax.nn.sigmoid(gi[:, H:2 * H] + gh[:, H:2 * H])
    nn_ = jnp.tanh(gi[:, 2 * H:] + r * gh[:, 2 * H:])
    return (1.0 - z) * nn_ + z * hid


def _tc1_body(na_ref, l1w_ref, l1b_ref, w1aT_ref, g2T_ref, attr_ref,
              x0_ref, am_ref, sr_ref):
    x0 = _lk(na_ref[...] * l1w_ref[...] + l1b_ref[...])
    x0_ref[...] = x0
    am_ref[:, :H] = jnp.dot(x0, w1aT_ref[...], preferred_element_type=F32)
    am_ref[:, H:] = jnp.dot(x0, g2T_ref[...], preferred_element_type=F32)
    sr_ref[...] = jnp.dot(x0, attr_ref[...], preferred_element_type=F32)


def _tc1(na, l1w, l1b, w1aT, g2T, attr_col):
    full = lambda s: pl.BlockSpec(s, lambda i: (0,) * len(s))
    return pl.pallas_call(
        _tc1_body,
        grid=(NB,),
        in_specs=[
            pl.BlockSpec((RB, 1), lambda i: (i, 0)),
            full((1, H)), full((1, H)), full((H, H)), full((H, H)),
            full((H, 1)),
        ],
        out_specs=[
            pl.BlockSpec((RB, H), lambda i: (i, 0)),
            pl.BlockSpec((RB, 2 * H), lambda i: (i, 0)),
            pl.BlockSpec((RB, 1), lambda i: (i, 0)),
        ],
        out_shape=[
            jax.ShapeDtypeStruct((NP, H), F32),
            jax.ShapeDtypeStruct((NP, 2 * H), F32),
            jax.ShapeDtypeStruct((NP, 1), F32),
        ],
        compiler_params=pltpu.CompilerParams(
            dimension_semantics=("arbitrary",)),
    )(na, l1w, l1b, w1aT, g2T, attr_col)


def _tc2_body(hacc_ref, s_ref, x0_ref, gb_ref, wihT_ref, whhT_ref, bih_ref,
              bhh_ref, awT_ref, asrc_ref, adst_ref,
              x1_ref, xs_ref, ssrc_ref, sdst_ref):
    hs = hacc_ref[0] + hacc_ref[1]
    ssum = s_ref[0] + s_ref[1]
    h = _elu(hs / (ssum + 1e-16) + gb_ref[...])
    x0 = x0_ref[...]
    x1 = jnp.maximum(
        _gru_block(h, x0, wihT_ref[...], whhT_ref[...], bih_ref[...],
                   bhh_ref[...]), 0.0)
    x1_ref[...] = x1
    xs = jnp.dot(x1, awT_ref[...], preferred_element_type=F32)
    xs_ref[...] = xs
    ssrc_ref[...] = jnp.dot(xs, asrc_ref[...], preferred_element_type=F32)
    sdst_ref[...] = jnp.dot(xs, adst_ref[...], preferred_element_type=F32)


def _tc2(hacc, s3, x0, gb, wihT, whhT, bih, bhh, awT, asrc_col, adst_col):
    full = lambda s: pl.BlockSpec(s, lambda i: (0,) * len(s))
    return pl.pallas_call(
        _tc2_body,
        grid=(NB,),
        in_specs=[
            pl.BlockSpec((2, RB, H), lambda i: (0, i, 0)),
            pl.BlockSpec((2, RB, 1), lambda i: (0, i, 0)),
            pl.BlockSpec((RB, H), lambda i: (i, 0)),
            full((1, H)), full((H, 3 * H)), full((H, 3 * H)),
            full((1, 3 * H)), full((1, 3 * H)), full((H, H)),
            full((H, 1)), full((H, 1)),
        ],
        out_specs=[
            pl.BlockSpec((RB, H), lambda i: (i, 0)),
            pl.BlockSpec((RB, H), lambda i: (i, 0)),
            pl.BlockSpec((RB, 1), lambda i: (i, 0)),
            pl.BlockSpec((RB, 1), lambda i: (i, 0)),
        ],
        out_shape=[
            jax.ShapeDtypeStruct((NP, H), F32),
            jax.ShapeDtypeStruct((NP, H), F32),
            jax.ShapeDtypeStruct((NP, 1), F32),
            jax.ShapeDtypeStruct((NP, 1), F32),
        ],
        compiler_params=pltpu.CompilerParams(
            dimension_semantics=("arbitrary",)),
    )(hacc, s3, x0, gb, wihT, whhT, bih, bhh, awT, asrc_col, adst_col)


def _tc3a_body(hacc_ref, s_ref, x1_ref, ab_ref, wihT_ref, whhT_ref, bih_ref,
               bhh_ref, mwT_ref, msrc_ref,
               xm_ref, ssrcm_ref, x2sum_ref, maxs_ref, acc_ref, mx_ref):
    i = pl.program_id(0)

    @pl.when(i == 0)
    def _():
        acc_ref[...] = jnp.zeros_like(acc_ref)
        mx_ref[0, 0] = -1e30

    hs = hacc_ref[0] + hacc_ref[1]
    ssum = s_ref[0] + s_ref[1]
    h = _elu(hs / (ssum + 1e-16) + ab_ref[...])
    x1 = x1_ref[...]
    x2 = jnp.maximum(
        _gru_block(h, x1, wihT_ref[...], whhT_ref[...], bih_ref[...],
                   bhh_ref[...]), 0.0)
    xm = jnp.dot(x2, mwT_ref[...], preferred_element_type=F32)
    xm_ref[...] = xm
    sm = jnp.dot(xm, msrc_ref[...], preferred_element_type=F32)
    rows = i * RB + lax.broadcasted_iota(jnp.int32, (RB, 1), 0)
    valid = rows < N
    sm = jnp.where(valid, sm, -1e30)
    ssrcm_ref[...] = sm
    acc_ref[...] += jnp.sum(jnp.where(valid, x2, 0.0), axis=0, keepdims=True)
    mx_ref[0, 0] = jnp.maximum(mx_ref[0, 0], jnp.max(sm))
    x2sum_ref[...] = acc_ref[...]
    maxs_ref[0, 0] = mx_ref[0, 0]


def _tc3a(hacc, s3, x1, ab, wihT, whhT, bih, bhh, mwT, msrc_col):
    full = lambda s: pl.BlockSpec(s, lambda i: (0,) * len(s))
    return pl.pallas_call(
        _tc3a_body,
        grid=(NB,),
        in_specs=[
            pl.BlockSpec((2, RB, H), lambda i: (0, i, 0)),
            pl.BlockSpec((2, RB, 1), lambda i: (0, i, 0)),
            pl.BlockSpec((RB, H), lambda i: (i, 0)),
            full((1, H)), full((H, 3 * H)), full((H, 3 * H)),
            full((1, 3 * H)), full((1, 3 * H)), full((H, H)),
            full((H, 1)),
        ],
        out_specs=[
            pl.BlockSpec((RB, H), lambda i: (i, 0)),
            pl.BlockSpec((RB, 1), lambda i: (i, 0)),
            pl.BlockSpec((1, H), lambda i: (0, 0)),
            pl.BlockSpec((1, 1), lambda i: (0, 0),
                         memory_space=pltpu.SMEM),
        ],
        out_shape=[
            jax.ShapeDtypeStruct((NP, H), F32),
            jax.ShapeDtypeStruct((NP, 1), F32),
            jax.ShapeDtypeStruct((1, H), F32),
            jax.ShapeDtypeStruct((1, 1), F32),
        ],
        scratch_shapes=[
            pltpu.VMEM((1, H), F32),
            pltpu.SMEM((1, 1), F32),
        ],
        compiler_params=pltpu.CompilerParams(
            dimension_semantics=("arbitrary",)),
    )(hacc, s3, x1, ab, wihT, whhT, bih, bhh, mwT, msrc_col)


def _tc3b_body(xm_ref, sm_ref, x2sum_ref, maxs_ref, mwT_ref, mdst_ref,
               mb_ref, wihT_ref, whhT_ref, bih_ref, bhh_ref, l2T_ref, l2b_ref,
               out_ref, sw_ref, wxm_ref):
    i = pl.program_id(0)

    @pl.when(i == 0)
    def _():
        sw_ref[0, 0] = 0.0
        wxm_ref[...] = jnp.zeros_like(wxm_ref)

    out0 = jnp.maximum(x2sum_ref[...], 0.0)
    cm = jnp.dot(out0, mwT_ref[...], preferred_element_type=F32)
    c = jnp.sum(cm * mdst_ref[...])
    mx = _lk(maxs_ref[0, 0] + c)
    am = _lk(sm_ref[...] + c)
    w = jnp.exp(am - mx)
    sw_ref[0, 0] += jnp.sum(w)
    wxm_ref[...] += jnp.sum(w * xm_ref[...], axis=0, keepdims=True)
    h3 = _elu(wxm_ref[...] / (sw_ref[0, 0] + 1e-16) + mb_ref[...])
    og = jnp.maximum(
        _gru_block(h3, out0, wihT_ref[...], whhT_ref[...], bih_ref[...],
                   bhh_ref[...]), 0.0)
    out_ref[...] = jnp.dot(og, l2T_ref[...], preferred_element_type=F32) \
        + l2b_ref[...]


def _tc3b(xm, ssrcm, x2sum, maxs, mwT, mdst_row, mb, wihT, whhT, bih, bhh,
          l2T, l2b):
    full = lambda s: pl.BlockSpec(s, lambda i: (0,) * len(s))
    return pl.pallas_call(
        _tc3b_body,
        grid=(NB,),
        in_specs=[
            pl.BlockSpec((RB, H), lambda i: (i, 0)),
            pl.BlockSpec((RB, 1), lambda i: (i, 0)),
            full((1, H)),
            pl.BlockSpec((1, 1), lambda i: (0, 0),
                         memory_space=pltpu.SMEM),
            full((H, H)), full((1, H)), full((1, H)),
            full((H, 3 * H)), full((H, 3 * H)),
            full((1, 3 * H)), full((1, 3 * H)),
            full((H, H)), full((1, H)),
        ],
        out_specs=pl.BlockSpec((1, H), lambda i: (0, 0)),
        out_shape=jax.ShapeDtypeStruct((1, H), F32),
        scratch_shapes=[
            pltpu.SMEM((1, 1), F32),
            pltpu.VMEM((1, H), F32),
        ],
        compiler_params=pltpu.CompilerParams(
            dimension_semantics=("arbitrary",)),
    )(xm, ssrcm, x2sum, maxs, mwT, mdst_row, mb, wihT, whhT, bih, bhh,
      l2T, l2b)


_MESH = dict(core_axis_name="c", subcore_axis_name="s",
             num_cores=2, num_subcores=16)


def _zero_rows(rows_ref, nrows):
    def zrow(e, _):
        for kk in range(H // 16):
            rows_ref[e, pl.ds(kk * 16, 16)] = jnp.zeros((16,), F32)
        return 0
    lax.fori_loop(0, nrows, zrow, 0)


def _prologue(rows_ref, s_part, hacc, s_sp, sid, c):
    _zero_rows(rows_ref, c)
    _zero_rows(s_part, NS)

    def zh(j, _):
        pltpu.sync_copy(rows_ref, hacc.at[pl.ds(sid * NPT + j * c, c)])
        return 0
    lax.fori_loop(0, NPT // c, zh, 0)

    @pl.when(sid < NS // 8)
    def _():
        pltpu.sync_copy(rows_ref.at[pl.ds(0, 8)], s_sp.at[pl.ds(sid * 8, 8)])


def _epilogue(sid, cid, s_part, hacc, s_sp, hacc_out, s_out, c):
    lane = lax.iota(jnp.int32, 16)
    for j in range(NS // 16):
        pltpu.sync_copy(s_part.at[pl.ds(j * 16, 16)], s_sp.at[lane + j * 16],
                        add=True)
    plsc.subcore_barrier()

    def wh(j, _):
        sl = pl.ds(sid * NPT + j * c, c)
        pltpu.sync_copy(hacc.at[sl], hacc_out.at[cid].at[sl])
        return 0
    lax.fori_loop(0, NPT // c, wh, 0)

    @pl.when(sid < NS // 8)
    def _():
        ssl = pl.ds(sid * 8, 8)
        pltpu.sync_copy(s_sp.at[ssl], s_out.at[cid].at[ssl])


def _sc1_body(am_hbm, sr_hbm, w1e_hbm, attl_hbm, src_hbm, dst_hbm,
              ea_hbm, hacc_out, s_out, hacc, s_sp, w1ev, attlv,
              s_part, srcb, dstb, eab, srbuf, amrows, scat,
              tbuf, sem_a, sem_sr, sem_sc):
    cid = lax.axis_index("c")
    sid = lax.axis_index("s")
    wid = cid * 16 + sid
    pltpu.sync_copy(w1e_hbm, w1ev)
    pltpu.sync_copy(attl_hbm, attlv)
    _prologue(scat.at[0], s_part, hacc, s_sp, sid, C1)
    plsc.subcore_barrier()
    w1 = [w1ev[pl.ds(kk * 16, 16)] for kk in range(8)]
    al = [attlv[pl.ds(kk * 16, 16)] for kk in range(8)]
    lane = lax.iota(jnp.int32, 16)

    def load_batch(j, jb):
        row = wid * CPW1 + j * BSZ
        sl = pl.ds(jb * BSZ, BSZ)
        pltpu.sync_copy(src_hbm.at[pl.ds(row, BSZ)], srcb.at[sl])
        pltpu.sync_copy(dst_hbm.at[pl.ds(row, BSZ)], dstb.at[sl])
        pltpu.sync_copy(ea_hbm.at[pl.ds(row, BSZ)], eab.at[sl])

    def issue(kn):
        jbn = (kn // BSZ) & 1
        rown = jbn * BSZ + kn % BSZ
        bn = kn & 1
        pltpu.async_copy(sr_hbm.at[dstb.at[rown]], srbuf.at[bn],
                         sem_sr.at[bn])
        pltpu.async_copy(am_hbm.at[srcb.at[rown]], amrows.at[bn],
                         sem_a.at[bn])

        @pl.when(kn >= 3)
        def _():
            pltpu.make_async_copy(scat.at[kn % 3], hacc.at[dstb.at[0]],
                                  sem_sc.at[kn % 3]).wait()

    load_batch(0, 0)
    issue(0)

    def chunk(k, _):
        b = k & 1
        b3 = k % 3
        row = ((k // BSZ) & 1) * BSZ + k % BSZ
        kn = k + 1

        @pl.when(kn < CPW1)
        def _():
            @pl.when(kn % BSZ == 0)
            def _():
                load_batch(kn // BSZ, (kn // BSZ) & 1)
            issue(kn)

        pltpu.make_async_copy(am_hbm.at[srcb.at[row]], amrows.at[b],
                              sem_a.at[b]).wait()
        pltpu.make_async_copy(sr_hbm.at[dstb.at[row]], srbuf.at[b],
                              sem_sr.at[b]).wait()

        @plsc.parallel_loop(0, C1 // 16, 1, unroll=4)
        def grp_t(g):
            ea16 = eab[row, pl.ds(g * 16, 16)]
            t16 = jnp.zeros((16,), F32)
            for e16 in range(16):
                e = g * 16 + e16
                ea_e = ea16[e16]
                acc = jnp.zeros((16,), F32)
                for kk in range(8):
                    v = amrows[b, e, pl.ds(kk * 16, 16)] + ea_e * w1[kk]
                    v = jnp.maximum(v, 0.01 * v)
                    acc = acc + al[kk] * v
                t16 = jnp.where(lane == e16, jnp.sum(acc), t16)
            dst16 = dstb[row, pl.ds(g * 16, 16)]
            t16 = t16 + srbuf[b, pl.ds(g * 16, 16)]
            t16 = jnp.maximum(t16, 0.01 * t16)
            ee16 = jnp.exp(t16)
            tbuf[pl.ds(g * 16, 16)] = ee16
            plsc.addupdate_scatter(s_part, [dst16 >> 7, dst16 & 127], ee16)

        @plsc.parallel_loop(0, C1 // 16, 1, unroll=4)
        def grp_s(g):
            ee16 = tbuf[pl.ds(g * 16, 16)]
            for e16 in range(16):
                e = g * 16 + e16
                ee_e = ee16[e16]
                for kk in range(8):
                    scat[b3, e, pl.ds(kk * 16, 16)] = \
                        amrows[b, e, pl.ds(H + kk * 16, 16)] * ee_e
        pltpu.async_copy(scat.at[b3], hacc.at[dstb.at[row]], sem_sc.at[b3],
                         add=True)
        return 0
    lax.fori_loop(0, CPW1, chunk, 0)
    for j in range(3):
        pltpu.make_async_copy(scat.at[(CPW1 - 3 + j) % 3],
                              hacc.at[dstb.at[0]],
                              sem_sc.at[(CPW1 - 3 + j) % 3]).wait()
    plsc.subcore_barrier()
    _epilogue(sid, cid, s_part, hacc, s_sp, hacc_out, s_out, C1)


@functools.cache
def _sc1_built():
    return pl.kernel(
        _sc1_body,
        out_type=[
            jax.ShapeDtypeStruct((2, NP, H), F32),
            jax.ShapeDtypeStruct((2, NS, 128), F32),
        ],
        mesh=plsc.VectorSubcoreMesh(**_MESH),
        scratch_types=[
            pltpu.VMEM_SHARED((NP, H), F32),
            pltpu.VMEM_SHARED((NS, 128), F32),
            pltpu.VMEM((H,), F32),
            pltpu.VMEM((H,), F32),
            pltpu.VMEM((NS, 128), F32),
            pltpu.VMEM((2 * BSZ, C1), jnp.int32),
            pltpu.VMEM((2 * BSZ, C1), jnp.int32),
            pltpu.VMEM((2 * BSZ, C1), F32),
            pltpu.VMEM((2, C1), F32),
            pltpu.VMEM((2, C1, 2 * H), F32),
            pltpu.VMEM((3, C1, H), F32),
            pltpu.VMEM((C1,), F32),
            pltpu.SemaphoreType.DMA((2,)),
            pltpu.SemaphoreType.DMA((2,)),
            pltpu.SemaphoreType.DMA((3,)),
        ],
        compiler_params=pltpu.CompilerParams(needs_layout_passes=False),
    )


def _sc1(*args):
    return _sc1_built()(*args)


def _sc2_body(xs_hbm, ssrc_hbm, sdst_hbm, src_hbm, dst_hbm, hacc_out, s_out,
              hacc, s_sp, s_part, srcb, dstb, sabuf, sbbuf,
              xsrows, ebuf, sem_x, sem_a, sem_b, sem_sc):
    cid = lax.axis_index("c")
    sid = lax.axis_index("s")
    wid = cid * 16 + sid
    _prologue(xsrows.at[0], s_part, hacc, s_sp, sid, C2)
    plsc.subcore_barrier()

    def load_batch(j, jb):
        row = wid * CPW2 + j * BSZ
        sl = pl.ds(jb * BSZ, BSZ)
        pltpu.sync_copy(src_hbm.at[pl.ds(row, BSZ)], srcb.at[sl])
        pltpu.sync_copy(dst_hbm.at[pl.ds(row, BSZ)], dstb.at[sl])

    def issue(kn):
        rown = ((kn // BSZ) & 1) * BSZ + kn % BSZ
        bn = kn & 1
        bn3 = kn % 3
        pltpu.async_copy(ssrc_hbm.at[srcb.at[rown]], sabuf.at[bn],
                         sem_a.at[bn])
        pltpu.async_copy(sdst_hbm.at[dstb.at[rown]], sbbuf.at[bn],
                         sem_b.at[bn])

        @pl.when(kn >= 3)
        def _():
            pltpu.make_async_copy(xsrows.at[bn3], hacc.at[dstb.at[0]],
                                  sem_sc.at[bn3]).wait()
        pltpu.async_copy(xs_hbm.at[srcb.at[rown]], xsrows.at[bn3],
                         sem_x.at[bn3])

    load_batch(0, 0)
    issue(0)

    def chunk(k, _):
        b = k & 1
        b3 = k % 3
        row = ((k // BSZ) & 1) * BSZ + k % BSZ
        kn = k + 1

        @pl.when(kn < CPW2)
        def _():
            @pl.when(kn % BSZ == 0)
            def _():
                load_batch(kn // BSZ, (kn // BSZ) & 1)
            issue(kn)

        pltpu.make_async_copy(ssrc_hbm.at[srcb.at[row]], sabuf.at[b],
                              sem_a.at[b]).wait()
        pltpu.make_async_copy(sdst_hbm.at[dstb.at[row]], sbbuf.at[b],
                              sem_b.at[b]).wait()
        for g in range(C2 // 16):
            dst16 = dstb[row, pl.ds(g * 16, 16)]
            a16 = sabuf[b, pl.ds(g * 16, 16)] + sbbuf[b, pl.ds(g * 16, 16)]
            a16 = jnp.maximum(a16, 0.01 * a16)
            ee16 = jnp.exp(a16)
            ebuf[pl.ds(g * 16, 16)] = ee16
            plsc.addupdate_scatter(s_part, [dst16 >> 7, dst16 & 127], ee16)
        pltpu.make_async_copy(xs_hbm.at[srcb.at[row]], xsrows.at[b3],
                              sem_x.at[b3]).wait()

        @plsc.parallel_loop(0, C2 // 16, 1, unroll=4)
        def grp_s(g):
            ee16 = ebuf[pl.ds(g * 16, 16)]
            for e16 in range(16):
                e = g * 16 + e16
                ee_e = ee16[e16]
                for kk in range(8):
                    xsrows[b3, e, pl.ds(kk * 16, 16)] = \
                        xsrows[b3, e, pl.ds(kk * 16, 16)] * ee_e
        pltpu.async_copy(xsrows.at[b3], hacc.at[dstb.at[row]], sem_sc.at[b3],
                         add=True)
        return 0
    lax.fori_loop(0, CPW2, chunk, 0)
    for j in range(3):
        pltpu.make_async_copy(xsrows.at[(CPW2 - 3 + j) % 3],
                              hacc.at[dstb.at[0]],
                              sem_sc.at[(CPW2 - 3 + j) % 3]).wait()
    plsc.subcore_barrier()
    _epilogue(sid, cid, s_part, hacc, s_sp, hacc_out, s_out, C2)


@functools.cache
def _sc2_built():
    return pl.kernel(
        _sc2_body,
        out_type=[
            jax.ShapeDtypeStruct((2, NP, H), F32),
            jax.ShapeDtypeStruct((2, NS, 128), F32),
        ],
        mesh=plsc.VectorSubcoreMesh(**_MESH),
        scratch_types=[
            pltpu.VMEM_SHARED((NP, H), F32),
            pltpu.VMEM_SHARED((NS, 128), F32),
            pltpu.VMEM((NS, 128), F32),
            pltpu.VMEM((2 * BSZ, C2), jnp.int32),
            pltpu.VMEM((2 * BSZ, C2), jnp.int32),
            pltpu.VMEM((2, C2), F32),
            pltpu.VMEM((2, C2), F32),
            pltpu.VMEM((3, C2, H), F32),
            pltpu.VMEM((C2,), F32),
            pltpu.SemaphoreType.DMA((3,)),
            pltpu.SemaphoreType.DMA((2,)),
            pltpu.SemaphoreType.DMA((2,)),
            pltpu.SemaphoreType.DMA((3,)),
        ],
        compiler_params=pltpu.CompilerParams(needs_layout_passes=False),
    )


def _sc2(*args):
    return _sc2_built()(*args)


def kernel(node_attr, edge_index, edge_attr, params):
    p = params
    na = jnp.pad(node_attr, ((0, NP - N), (0, 0)))
    src_p = jnp.pad(edge_index[0], (0, EP - E), constant_values=NP - 1)
    dst_p = jnp.pad(edge_index[1], (0, EP - E), constant_values=NP - 1)
    ea_p = jnp.pad(edge_attr[:, 0], (0, EP - E))
    src2d1 = src_p.reshape(EP // C1, C1)
    dst2d1 = dst_p.reshape(EP // C1, C1)
    ea2d1 = ea_p.reshape(EP // C1, C1)
    src2d2 = src_p.reshape(EP // C2, C2)
    dst2d2 = dst_p.reshape(EP // C2, C2)

    l1w = p["lin1_w"].T
    l1b = p["lin1_b"].reshape(1, H)
    w1aT = p["gate_lin1_w"][:, :H].T
    w1e = p["gate_lin1_w"][:, H]
    g2T = p["gate_lin2_w"].T
    attr_col = p["gate_att_r"].reshape(H, 1)

    x0, am_mat, sr = _tc1(na, l1w, l1b, w1aT, g2T, attr_col)
    hacc, s1 = _sc1(am_mat, sr.reshape(NP), w1e, p["gate_att_l"],
                    src2d1, dst2d1, ea2d1)
    x1, xs, ssrc, sdst = _tc2(
        hacc, s1.reshape(2, NP, 1), x0, p["gate_bias"].reshape(1, H),
        p["gru1_wih"].T, p["gru1_whh"].T,
        p["gru1_bih"].reshape(1, 3 * H), p["gru1_bhh"].reshape(1, 3 * H),
        p["atom_w"].T, p["atom_att_src"].reshape(H, 1),
        p["atom_att_dst"].reshape(H, 1))
    hacc2, s2 = _sc2(xs, ssrc.reshape(NP), sdst.reshape(NP), src2d2, dst2d2)
    xm, ssrcm, x2sum, maxs = _tc3a(
        hacc2, s2.reshape(2, NP, 1), x1, p["atom_bias"].reshape(1, H),
        p["gru2_wih"].T, p["gru2_whh"].T,
        p["gru2_bih"].reshape(1, 3 * H), p["gru2_bhh"].reshape(1, 3 * H),
        p["mol_w"].T, p["mol_att_src"].reshape(H, 1))
    out = _tc3b(
        xm, ssrcm, x2sum, maxs, p["mol_w"].T,
        p["mol_att_dst"].reshape(1, H), p["mol_bias"].reshape(1, H),
        p["grum_wih"].T, p["grum_whh"].T,
        p["grum_bih"].reshape(1, 3 * H), p["grum_bhh"].reshape(1, 3 * H),
        p["lin2_w"].T, p["lin2_b"].reshape(1, H))
    return out
